# Initial kernel scaffold; baseline (speedup 1.0000x reference)
import jax
import jax.numpy as jnp
from jax.experimental import pallas as pl


def kernel(x, edge_index, W1, b1, W2, b2):
    raise NotImplementedError("write your pallas kernel here")


# speedup vs baseline: 2.8195x; 2.8195x over previous
import dataclasses
import functools

import jax
import jax.numpy as jnp
from jax import lax
from jax.experimental import pallas as pl
from jax.experimental.pallas import tpu as pltpu
from jax.experimental.pallas import tpu_sc as plsc

N = 10000
NP = 10240
D = 128
E = 320000
ER = 2560
PAD_IDX = 10000
NC, NS = 2, 16
NW = NC * NS
ROWS_PER_TILE = ER // NW
GROUPS = ROWS_PER_TILE // 8
NODE_ROWS_PER_TILE = NP // NS
NBLK = 512
GRID = NP // NBLK


def _mesh():
    return plsc.VectorSubcoreMesh(core_axis_name="c", subcore_axis_name="s")


def _sc_compiler_params():
    cp = pltpu.CompilerParams()
    if "needs_layout_passes" in pltpu.CompilerParams.__dataclass_fields__:
        cp = dataclasses.replace(cp, needs_layout_passes=False)
    return cp


def _deg_body(srcr, dstr, out, acc, src8, dst8):
    c = lax.axis_index("c")
    s = lax.axis_index("s")
    w = c * NS + s
    z16 = jnp.zeros((16,), jnp.float32)

    @pl.loop(0, (2 * NP) // 16)
    def _(i):
        acc[pl.ds(i * 16, 16)] = z16

    ones = jnp.ones((16,), jnp.float32)
    base = w * ROWS_PER_TILE

    @pl.loop(0, GROUPS)
    def _(g):
        r0 = base + g * 8
        pltpu.sync_copy(srcr.at[pl.ds(r0, 8)], src8)
        pltpu.sync_copy(dstr.at[pl.ds(r0, 8)], dst8)
        for j in range(8):
            for k in range(8):
                iv_s = src8[j, pl.ds(k * 16, 16)]
                plsc.addupdate_scatter(acc, [iv_s * 2], ones)
                iv_d = dst8[j, pl.ds(k * 16, 16)]
                plsc.addupdate_scatter(acc, [iv_d * 2 + 1], ones)

    pltpu.sync_copy(acc, out.at[w])


def _make_deg_kernel():
    return pl.kernel(
        _deg_body,
        out_type=jax.ShapeDtypeStruct((NW, 2 * NP), jnp.float32),
        mesh=_mesh(),
        compiler_params=_sc_compiler_params(),
        scratch_types=[
            pltpu.VMEM((2 * NP,), jnp.float32),
            pltpu.VMEM((8, 128), jnp.int32),
            pltpu.VMEM((8, 128), jnp.int32),
        ],
    )


def _seg_body(h, srcr, dstr, out, acc, rows, src8, dst8, zb, sem0, sem1):
    c = lax.axis_index("c")
    s = lax.axis_index("s")
    w = c * NS + s
    z16 = jnp.zeros((16,), jnp.float32)

    @pl.loop(0, 64)
    def _(i):
        for k in range(8):
            zb[i, pl.ds(k * 16, 16)] = z16

    node0 = s * NODE_ROWS_PER_TILE
    for t in range(NODE_ROWS_PER_TILE // 64):
        pltpu.sync_copy(zb, acc.at[pl.ds(node0 + t * 64, 64)])
    plsc.subcore_barrier()

    base = w * ROWS_PER_TILE
    sems = (sem0, sem1)

    @pl.loop(0, GROUPS)
    def _(g):
        r0 = base + g * 8
        pltpu.sync_copy(srcr.at[pl.ds(r0, 8)], src8)
        pltpu.sync_copy(dstr.at[pl.ds(r0, 8)], dst8)
        prev = None
        for j in range(8):
            d = pltpu.async_copy(h.at[src8.at[j]], rows.at[j % 2], sems[j % 2])
            if prev is not None:
                prev.wait()
                pltpu.sync_copy(rows.at[(j - 1) % 2], acc.at[dst8.at[j - 1]],
                                add=True)
            prev = d
        prev.wait()
        pltpu.sync_copy(rows.at[7 % 2], acc.at[dst8.at[7]], add=True)

    plsc.subcore_barrier()
    for t in range(NODE_ROWS_PER_TILE // 128):
        sl = pl.ds(node0 + t * 128, 128)
        pltpu.sync_copy(acc.at[sl], out.at[c, sl])


def _make_seg_kernel():
    return pl.kernel(
        _seg_body,
        out_type=jax.ShapeDtypeStruct((NC, NP, D), jnp.float32),
        mesh=_mesh(),
        scratch_types=[
            pltpu.VMEM_SHARED((NP, D), jnp.float32),
            pltpu.VMEM((2, 128, D), jnp.float32),
            pltpu.VMEM((8, 128), jnp.int32),
            pltpu.VMEM((8, 128), jnp.int32),
            pltpu.VMEM((64, D), jnp.float32),
            pltpu.SemaphoreType.DMA,
            pltpu.SemaphoreType.DMA,
        ],
    )


def _scales(cnt_blk):
    deg = jnp.maximum(jnp.sum(cnt_blk, axis=0), 1.0)
    return lax.rsqrt(deg[:, 0:1]), lax.rsqrt(deg[:, 1:2])


def _prep_body(x_ref, cnt_ref, o_ref):
    so, _ = _scales(cnt_ref[...])
    o_ref[...] = x_ref[...] * so


_prep_call = pl.pallas_call(
    _prep_body,
    grid=(GRID,),
    in_specs=[
        pl.BlockSpec((NBLK, D), lambda i: (i, 0)),
        pl.BlockSpec((NW, NBLK, 2), lambda i: (0, i, 0)),
    ],
    out_specs=pl.BlockSpec((NBLK, D), lambda i: (i, 0)),
    out_shape=jax.ShapeDtypeStruct((NP, D), jnp.float32),
)


def _combine_body(parts_ref, cnt_ref, w_ref, b_ref, o_ref, *, layer1):
    so, si = _scales(cnt_ref[...])
    pp = parts_ref[...]
    p = (pp[0] + pp[1]) * si
    z = lax.dot_general(p, w_ref[...], (((1,), (0,)), ((), ())),
                        preferred_element_type=jnp.float32,
                        precision=lax.Precision.HIGHEST)
    z = z + b_ref[...]
    if layer1:
        z = jnp.maximum(z, 0.0) * so
    o_ref[...] = z


def _make_combine(layer1):
    return pl.pallas_call(
        functools.partial(_combine_body, layer1=layer1),
        grid=(GRID,),
        in_specs=[
            pl.BlockSpec((NC, NBLK, D), lambda i: (0, i, 0)),
            pl.BlockSpec((NW, NBLK, 2), lambda i: (0, i, 0)),
            pl.BlockSpec((D, D), lambda i: (0, 0)),
            pl.BlockSpec((1, D), lambda i: (0, 0)),
        ],
        out_specs=pl.BlockSpec((NBLK, D), lambda i: (i, 0)),
        out_shape=jax.ShapeDtypeStruct((NP, D), jnp.float32),
    )


def kernel(x, edge_index, W1, b1, W2, b2):
    src = edge_index[0]
    dst = edge_index[1]
    pad = jnp.full((ER * 128 - E,), PAD_IDX, jnp.int32)
    srcr = jnp.concatenate([src, pad]).reshape(ER, 128)
    dstr = jnp.concatenate([dst, pad]).reshape(ER, 128)
    x_pad = jnp.pad(x, ((0, NP - N), (0, 0)))
    b1r = b1.reshape(1, D)
    b2r = b2.reshape(1, D)

    cnt = _make_deg_kernel()(srcr, dstr).reshape(NW, NP, 2)
    h1 = _prep_call(x_pad, cnt)
    seg = _make_seg_kernel()
    parts1 = seg(h1, srcr, dstr)
    h2 = _make_combine(True)(parts1, cnt, W1, b1r)
    parts2 = seg(h2, srcr, dstr)
    out = _make_combine(False)(parts2, cnt, W2, b2r)
    return out[:N]


# speedup vs baseline: 9.8797x; 3.5041x over previous
import dataclasses
import functools

import jax
import jax.numpy as jnp
from jax import lax
from jax.experimental import pallas as pl
from jax.experimental.pallas import tpu as pltpu
from jax.experimental.pallas import tpu_sc as plsc

N = 10000
NP = 10240
D = 128
E = 320000
ER = 2560
NC, NS = 2, 16
NW = NC * NS
ROWS_PER_TILE = ER // NW
GROUPS = ROWS_PER_TILE // 8
NODE_ROWS_PER_TILE = NP // NS
NBLK = 512
GRID = NP // NBLK


def _mesh():
    return plsc.VectorSubcoreMesh(core_axis_name="c", subcore_axis_name="s")


def _sc_compiler_params():
    cp = pltpu.CompilerParams()
    if "needs_layout_passes" in pltpu.CompilerParams.__dataclass_fields__:
        cp = dataclasses.replace(cp, needs_layout_passes=False)
    return cp


def _deg_body(srcr, dstr, out, acc_s, acc_d, src8, dst8):
    c = lax.axis_index("c")
    s = lax.axis_index("s")
    w = c * NS + s
    z16 = jnp.zeros((16,), jnp.float32)

    @pl.loop(0, NP // 16)
    def _(i):
        acc_s[pl.ds(i * 16, 16)] = z16
        acc_d[pl.ds(i * 16, 16)] = z16

    ones = jnp.ones((16,), jnp.float32)
    base = w * ROWS_PER_TILE

    @pl.loop(0, GROUPS)
    def _(g):
        r0 = base + g * 8
        pltpu.sync_copy(srcr.at[pl.ds(r0, 8)], src8)
        pltpu.sync_copy(dstr.at[pl.ds(r0, 8)], dst8)
        for j in range(8):
            for k in range(8):
                iv_s = src8[j, pl.ds(k * 16, 16)]
                plsc.addupdate_scatter(acc_s, [iv_s], ones)
                iv_d = dst8[j, pl.ds(k * 16, 16)]
                plsc.addupdate_scatter(acc_d, [iv_d], ones)

    pltpu.sync_copy(acc_s, out.at[0, w])
    pltpu.sync_copy(acc_d, out.at[1, w])


def _make_deg_kernel():
    return pl.kernel(
        _deg_body,
        out_type=jax.ShapeDtypeStruct((2, NW, NP), jnp.float32),
        mesh=_mesh(),
        compiler_params=_sc_compiler_params(),
        scratch_types=[
            pltpu.VMEM((NP,), jnp.float32),
            pltpu.VMEM((NP,), jnp.float32),
            pltpu.VMEM((8, 128), jnp.int32),
            pltpu.VMEM((8, 128), jnp.int32),
        ],
    )


def _seg_body(h, srcr, dstr, out, acc, rows, src8, dst8, zb, sem0, sem1):
    c = lax.axis_index("c")
    s = lax.axis_index("s")
    w = c * NS + s
    z16 = jnp.zeros((16,), jnp.float32)

    @pl.loop(0, 64)
    def _(i):
        for k in range(8):
            zb[i, pl.ds(k * 16, 16)] = z16

    node0 = s * NODE_ROWS_PER_TILE
    for t in range(NODE_ROWS_PER_TILE // 64):
        pltpu.sync_copy(zb, acc.at[pl.ds(node0 + t * 64, 64)])
    plsc.subcore_barrier()

    base = w * ROWS_PER_TILE
    sems = (sem0, sem1)

    @pl.loop(0, GROUPS)
    def _(g):
        r0 = base + g * 8
        pltpu.sync_copy(srcr.at[pl.ds(r0, 8)], src8)
        pltpu.sync_copy(dstr.at[pl.ds(r0, 8)], dst8)
        prev = None
        for j in range(8):
            d = pltpu.async_copy(h.at[src8.at[j]], rows.at[j % 2], sems[j % 2])
            if prev is not None:
                prev.wait()
                pltpu.sync_copy(rows.at[(j - 1) % 2], acc.at[dst8.at[j - 1]],
                                add=True)
            prev = d
        prev.wait()
        pltpu.sync_copy(rows.at[7 % 2], acc.at[dst8.at[7]], add=True)

    plsc.subcore_barrier()
    for t in range(NODE_ROWS_PER_TILE // 128):
        sl = pl.ds(node0 + t * 128, 128)
        pltpu.sync_copy(acc.at[sl], out.at[c, sl])


def _make_seg_kernel():
    return pl.kernel(
        _seg_body,
        out_type=jax.ShapeDtypeStruct((NC, NP, D), jnp.float32),
        mesh=_mesh(),
        compiler_params=_sc_compiler_params(),
        scratch_types=[
            pltpu.VMEM_SHARED((NP, D), jnp.float32),
            pltpu.VMEM((2, 128, D), jnp.float32),
            pltpu.VMEM((8, 128), jnp.int32),
            pltpu.VMEM((8, 128), jnp.int32),
            pltpu.VMEM((64, D), jnp.float32),
            pltpu.SemaphoreType.DMA,
            pltpu.SemaphoreType.DMA,
        ],
    )


def _scales_body(cnt_ref, so_ref, si_ref):
    cnt = cnt_ref[...]
    deg = jnp.maximum(jnp.sum(cnt, axis=1), 1.0)
    r = lax.rsqrt(deg)
    eye = (lax.broadcasted_iota(jnp.int32, (NBLK, NBLK), 0)
           == lax.broadcasted_iota(jnp.int32, (NBLK, NBLK), 1)
           ).astype(jnp.float32)
    ones = jnp.ones((NBLK, D), jnp.float32)
    dn = (((1,), (0,)), ((), ()))
    so_ref[...] = lax.dot_general(eye * r[0:1, :], ones, dn,
                                  preferred_element_type=jnp.float32,
                                  precision=lax.Precision.HIGHEST)
    si_ref[...] = lax.dot_general(eye * r[1:2, :], ones, dn,
                                  preferred_element_type=jnp.float32,
                                  precision=lax.Precision.HIGHEST)


_scales_call = pl.pallas_call(
    _scales_body,
    grid=(GRID,),
    in_specs=[pl.BlockSpec((2, NW, NBLK), lambda i: (0, 0, i))],
    out_specs=[pl.BlockSpec((NBLK, D), lambda i: (i, 0)),
               pl.BlockSpec((NBLK, D), lambda i: (i, 0))],
    out_shape=[jax.ShapeDtypeStruct((NP, D), jnp.float32),
               jax.ShapeDtypeStruct((NP, D), jnp.float32)],
)


def _prep_body(x_ref, so_ref, o_ref):
    o_ref[...] = x_ref[...] * so_ref[...]


_prep_call = pl.pallas_call(
    _prep_body,
    grid=(GRID,),
    in_specs=[
        pl.BlockSpec((NBLK, D), lambda i: (i, 0)),
        pl.BlockSpec((NBLK, D), lambda i: (i, 0)),
    ],
    out_specs=pl.BlockSpec((NBLK, D), lambda i: (i, 0)),
    out_shape=jax.ShapeDtypeStruct((NP, D), jnp.float32),
)


def _combine1_body(parts_ref, si_ref, so_ref, w_ref, b_ref, o_ref):
    pp = parts_ref[...]
    p = (pp[0] + pp[1]) * si_ref[...]
    z = lax.dot_general(p, w_ref[...], (((1,), (0,)), ((), ())),
                        preferred_element_type=jnp.float32,
                        precision=lax.Precision.HIGHEST)
    z = z + b_ref[...]
    o_ref[...] = jnp.maximum(z, 0.0) * so_ref[...]


def _combine2_body(parts_ref, si_ref, w_ref, b_ref, o_ref):
    pp = parts_ref[...]
    p = (pp[0] + pp[1]) * si_ref[...]
    z = lax.dot_general(p, w_ref[...], (((1,), (0,)), ((), ())),
                        preferred_element_type=jnp.float32,
                        precision=lax.Precision.HIGHEST)
    o_ref[...] = z + b_ref[...]


_nd_spec = pl.BlockSpec((NBLK, D), lambda i: (i, 0))

_combine1_call = pl.pallas_call(
    _combine1_body,
    grid=(GRID,),
    in_specs=[
        pl.BlockSpec((NC, NBLK, D), lambda i: (0, i, 0)),
        _nd_spec,
        _nd_spec,
        pl.BlockSpec((D, D), lambda i: (0, 0)),
        pl.BlockSpec((1, D), lambda i: (0, 0)),
    ],
    out_specs=_nd_spec,
    out_shape=jax.ShapeDtypeStruct((NP, D), jnp.float32),
)

_combine2_call = pl.pallas_call(
    _combine2_body,
    grid=(GRID,),
    in_specs=[
        pl.BlockSpec((NC, NBLK, D), lambda i: (0, i, 0)),
        _nd_spec,
        pl.BlockSpec((D, D), lambda i: (0, 0)),
        pl.BlockSpec((1, D), lambda i: (0, 0)),
    ],
    out_specs=_nd_spec,
    out_shape=jax.ShapeDtypeStruct((NP, D), jnp.float32),
)


def kernel(x, edge_index, W1, b1, W2, b2):
    src = edge_index[0]
    dst = edge_index[1]
    padlen = ER * 128 - E
    padidx = (N + (jnp.arange(padlen, dtype=jnp.int32) % (NP - N))
              ).astype(jnp.int32)
    srcr = jnp.concatenate([src, padidx]).reshape(ER, 128)
    dstr = jnp.concatenate([dst, padidx]).reshape(ER, 128)
    x_pad = jnp.pad(x, ((0, NP - N), (0, 0)))
    b1r = b1.reshape(1, D)
    b2r = b2.reshape(1, D)

    cnt = _make_deg_kernel()(srcr, dstr)
    so, si = _scales_call(cnt)
    h1 = _prep_call(x_pad, so)
    seg = _make_seg_kernel()
    parts1 = seg(h1, srcr, dstr)
    h2 = _combine1_call(parts1, si, so, W1, b1r)
    parts2 = seg(h2, srcr, dstr)
    out = _combine2_call(parts2, si, W2, b2r)
    return out[:N]


# speedup vs baseline: 11.4780x; 1.1618x over previous
import dataclasses
import functools

import jax
import jax.numpy as jnp
from jax import lax
from jax.experimental import pallas as pl
from jax.experimental.pallas import tpu as pltpu
from jax.experimental.pallas import tpu_sc as plsc

N = 10000
NP = 10240
D = 128
E = 320000
ER = 2560
NC, NS = 2, 16
NW = NC * NS
ROWS_PER_TILE = ER // NW
GROUPS = ROWS_PER_TILE // 8
NODE_ROWS_PER_TILE = NP // NS
NBLK = 512
GRID = NP // NBLK


def _mesh():
    return plsc.VectorSubcoreMesh(core_axis_name="c", subcore_axis_name="s")


def _sc_compiler_params():
    cp = pltpu.CompilerParams()
    if "needs_layout_passes" in pltpu.CompilerParams.__dataclass_fields__:
        cp = dataclasses.replace(cp, needs_layout_passes=False)
    return cp


def _deg_body(srcr, dstr, out, acc_s, acc_d, src8, dst8):
    c = lax.axis_index("c")
    s = lax.axis_index("s")
    w = c * NS + s
    z16 = jnp.zeros((16,), jnp.float32)

    @pl.loop(0, NP // 16)
    def _(i):
        acc_s[pl.ds(i * 16, 16)] = z16
        acc_d[pl.ds(i * 16, 16)] = z16

    ones = jnp.ones((16,), jnp.float32)
    base = w * ROWS_PER_TILE

    @pl.loop(0, GROUPS)
    def _(g):
        r0 = base + g * 8
        pltpu.sync_copy(srcr.at[pl.ds(r0, 8)], src8)
        pltpu.sync_copy(dstr.at[pl.ds(r0, 8)], dst8)
        for j in range(8):
            for k in range(8):
                iv_s = src8[j, pl.ds(k * 16, 16)]
                plsc.addupdate_scatter(acc_s, [iv_s], ones)
                iv_d = dst8[j, pl.ds(k * 16, 16)]
                plsc.addupdate_scatter(acc_d, [iv_d], ones)

    pltpu.sync_copy(acc_s, out.at[0, w])
    pltpu.sync_copy(acc_d, out.at[1, w])


def _make_deg_kernel():
    return pl.kernel(
        _deg_body,
        out_type=jax.ShapeDtypeStruct((2, NW, NP), jnp.float32),
        mesh=_mesh(),
        compiler_params=_sc_compiler_params(),
        scratch_types=[
            pltpu.VMEM((NP,), jnp.float32),
            pltpu.VMEM((NP,), jnp.float32),
            pltpu.VMEM((8, 128), jnp.int32),
            pltpu.VMEM((8, 128), jnp.int32),
        ],
    )


def _seg_body(h, srcr, dstr, out, acc, rows, idx, zb, sem0, sem1, isem0, isem1):
    c = lax.axis_index("c")
    s = lax.axis_index("s")
    w = c * NS + s
    z16 = jnp.zeros((16,), jnp.float32)

    @pl.loop(0, 64)
    def _(i):
        for k in range(8):
            zb[i, pl.ds(k * 16, 16)] = z16

    node0 = s * NODE_ROWS_PER_TILE
    for t in range(NODE_ROWS_PER_TILE // 64):
        pltpu.sync_copy(zb, acc.at[pl.ds(node0 + t * 64, 64)])
    plsc.subcore_barrier()

    base = w * ROWS_PER_TILE
    sems = (sem0, sem1)
    isems = (isem0, isem1)

    def fetch_idx(g, b):
        r0 = jnp.minimum(base + g * 8, ER - 8)
        return (pltpu.async_copy(srcr.at[pl.ds(r0, 8)], idx.at[b, 0], isems[b]),
                pltpu.async_copy(dstr.at[pl.ds(r0, 8)], idx.at[b, 1], isems[b]))

    for d0 in fetch_idx(0, 0):
        d0.wait()

    def process_group(g, b):
        nxt = fetch_idx(g + 1, 1 - b)
        prev = None
        for j in range(8):
            d = pltpu.async_copy(h.at[idx.at[b, 0, j]], rows.at[j % 2],
                                 sems[j % 2])
            if prev is not None:
                prev.wait()
                pltpu.sync_copy(rows.at[(j - 1) % 2],
                                acc.at[idx.at[b, 1, j - 1]], add=True)
            prev = d
        prev.wait()
        pltpu.sync_copy(rows.at[7 % 2], acc.at[idx.at[b, 1, 7]], add=True)
        for d1 in nxt:
            d1.wait()

    @pl.loop(0, GROUPS // 2)
    def _(t):
        process_group(2 * t, 0)
        process_group(2 * t + 1, 1)

    plsc.subcore_barrier()
    for t in range(NODE_ROWS_PER_TILE // 128):
        sl = pl.ds(node0 + t * 128, 128)
        pltpu.sync_copy(acc.at[sl], out.at[c, sl])


def _make_seg_kernel():
    return pl.kernel(
        _seg_body,
        out_type=jax.ShapeDtypeStruct((NC, NP, D), jnp.float32),
        mesh=_mesh(),
        compiler_params=_sc_compiler_params(),
        scratch_types=[
            pltpu.VMEM_SHARED((NP, D), jnp.float32),
            pltpu.VMEM((2, 128, D), jnp.float32),
            pltpu.VMEM((2, 2, 8, 128), jnp.int32),
            pltpu.VMEM((64, D), jnp.float32),
            pltpu.SemaphoreType.DMA,
            pltpu.SemaphoreType.DMA,
            pltpu.SemaphoreType.DMA,
            pltpu.SemaphoreType.DMA,
        ],
    )


def _scales_body(cnt_ref, x_ref, so_ref, si_ref, h1_ref):
    cnt = cnt_ref[...]
    deg = jnp.maximum(jnp.sum(cnt, axis=1), 1.0)
    rt = jnp.transpose(lax.rsqrt(deg))
    ones_row = jnp.ones((1, D), jnp.float32)
    so = rt[:, 0:1] * ones_row
    so_ref[...] = so
    si_ref[...] = rt[:, 1:2] * ones_row
    h1_ref[...] = x_ref[...] * so


_scales_call = pl.pallas_call(
    _scales_body,
    grid=(GRID,),
    in_specs=[pl.BlockSpec((2, NW, NBLK), lambda i: (0, 0, i)),
              pl.BlockSpec((NBLK, D), lambda i: (i, 0))],
    out_specs=[pl.BlockSpec((NBLK, D), lambda i: (i, 0)),
               pl.BlockSpec((NBLK, D), lambda i: (i, 0)),
               pl.BlockSpec((NBLK, D), lambda i: (i, 0))],
    out_shape=[jax.ShapeDtypeStruct((NP, D), jnp.float32),
               jax.ShapeDtypeStruct((NP, D), jnp.float32),
               jax.ShapeDtypeStruct((NP, D), jnp.float32)],
)


def _combine1_body(parts_ref, si_ref, so_ref, w_ref, b_ref, o_ref):
    pp = parts_ref[...]
    p = (pp[0] + pp[1]) * si_ref[...]
    z = lax.dot_general(p, w_ref[...], (((1,), (0,)), ((), ())),
                        preferred_element_type=jnp.float32,
                        precision=lax.Precision.HIGHEST)
    z = z + b_ref[...]
    o_ref[...] = jnp.maximum(z, 0.0) * so_ref[...]


def _combine2_body(parts_ref, si_ref, w_ref, b_ref, o_ref):
    pp = parts_ref[...]
    p = (pp[0] + pp[1]) * si_ref[...]
    z = lax.dot_general(p, w_ref[...], (((1,), (0,)), ((), ())),
                        preferred_element_type=jnp.float32,
                        precision=lax.Precision.HIGHEST)
    o_ref[...] = z + b_ref[...]


_nd_spec = pl.BlockSpec((NBLK, D), lambda i: (i, 0))

_combine1_call = pl.pallas_call(
    _combine1_body,
    grid=(GRID,),
    in_specs=[
        pl.BlockSpec((NC, NBLK, D), lambda i: (0, i, 0)),
        _nd_spec,
        _nd_spec,
        pl.BlockSpec((D, D), lambda i: (0, 0)),
        pl.BlockSpec((1, D), lambda i: (0, 0)),
    ],
    out_specs=_nd_spec,
    out_shape=jax.ShapeDtypeStruct((NP, D), jnp.float32),
)

NBLK2 = 400
_combine2_call = pl.pallas_call(
    _combine2_body,
    grid=(N // NBLK2,),
    in_specs=[
        pl.BlockSpec((NC, NBLK2, D), lambda i: (0, i, 0)),
        pl.BlockSpec((NBLK2, D), lambda i: (i, 0)),
        pl.BlockSpec((D, D), lambda i: (0, 0)),
        pl.BlockSpec((1, D), lambda i: (0, 0)),
    ],
    out_specs=pl.BlockSpec((NBLK2, D), lambda i: (i, 0)),
    out_shape=jax.ShapeDtypeStruct((N, D), jnp.float32),
)


def kernel(x, edge_index, W1, b1, W2, b2):
    src = edge_index[0]
    dst = edge_index[1]
    padlen = ER * 128 - E
    padidx = (N + (jnp.arange(padlen, dtype=jnp.int32) % (NP - N))
              ).astype(jnp.int32)
    srcr = jnp.concatenate([src, padidx]).reshape(ER, 128)
    dstr = jnp.concatenate([dst, padidx]).reshape(ER, 128)
    x_pad = jnp.pad(x, ((0, NP - N), (0, 0)))
    b1r = b1.reshape(1, D)
    b2r = b2.reshape(1, D)

    cnt = _make_deg_kernel()(srcr, dstr)
    so, si, h1 = _scales_call(cnt, x_pad)
    seg = _make_seg_kernel()
    parts1 = seg(h1, srcr, dstr)
    h2 = _combine1_call(parts1, si, so, W1, b1r)
    parts2 = seg(h2, srcr, dstr)
    return _combine2_call(parts2, si, W2, b2r)


# speedup vs baseline: 11.5745x; 1.0084x over previous
import dataclasses
import functools

import jax
import jax.numpy as jnp
from jax import lax
from jax.experimental import pallas as pl
from jax.experimental.pallas import tpu as pltpu
from jax.experimental.pallas import tpu_sc as plsc

N = 10000
NP = 10240
D = 128
E = 320000
ER = 2560
NC, NS = 2, 16
NW = NC * NS
ROWS_PER_TILE = ER // NW
GROUPS = ROWS_PER_TILE // 8
NODE_ROWS_PER_TILE = NP // NS
NBLK = 512
GRID = NP // NBLK


def _mesh():
    return plsc.VectorSubcoreMesh(core_axis_name="c", subcore_axis_name="s")


def _sc_compiler_params():
    cp = pltpu.CompilerParams()
    if "needs_layout_passes" in pltpu.CompilerParams.__dataclass_fields__:
        cp = dataclasses.replace(cp, needs_layout_passes=False)
    return cp


def _deg_body(srcr, dstr, out, acc_s, acc_d, src8, dst8):
    c = lax.axis_index("c")
    s = lax.axis_index("s")
    w = c * NS + s
    z16 = jnp.zeros((16,), jnp.float32)

    @pl.loop(0, NP // 16)
    def _(i):
        acc_s[pl.ds(i * 16, 16)] = z16
        acc_d[pl.ds(i * 16, 16)] = z16

    ones = jnp.ones((16,), jnp.float32)
    base = w * (ROWS_PER_TILE * 2)

    @pl.loop(0, GROUPS)
    def _(g):
        r0 = base + g * 16
        pltpu.sync_copy(srcr.at[pl.ds(r0, 16)], src8)
        pltpu.sync_copy(dstr.at[pl.ds(r0, 16)], dst8)
        for j in range(16):
            for k in range(4):
                iv_s = src8[j, pl.ds(k * 16, 16)]
                plsc.addupdate_scatter(acc_s, [iv_s], ones)
                iv_d = dst8[j, pl.ds(k * 16, 16)]
                plsc.addupdate_scatter(acc_d, [iv_d], ones)

    pltpu.sync_copy(acc_s, out.at[0, w])
    pltpu.sync_copy(acc_d, out.at[1, w])


def _make_deg_kernel():
    return pl.kernel(
        _deg_body,
        out_type=jax.ShapeDtypeStruct((2, NW, NP), jnp.float32),
        mesh=_mesh(),
        compiler_params=_sc_compiler_params(),
        scratch_types=[
            pltpu.VMEM((NP,), jnp.float32),
            pltpu.VMEM((NP,), jnp.float32),
            pltpu.VMEM((16, 64), jnp.int32),
            pltpu.VMEM((16, 64), jnp.int32),
        ],
    )


UNIT = 64
UNITS_PER_TILE = ROWS_PER_TILE * 128 // UNIT
GUNITS = 16
SEG_GROUPS = UNITS_PER_TILE // GUNITS
ERU = ER * 128 // UNIT
NSLOT = 4


def _seg_body(h, srcr, dstr, out, acc, rows, idx, zb,
              g0, g1, g2, g3, s0, s1, s2, s3, i0, i1):
    c = lax.axis_index("c")
    s = lax.axis_index("s")
    w = c * NS + s
    z16 = jnp.zeros((16,), jnp.float32)
    gsems = (g0, g1, g2, g3)
    ssems = (s0, s1, s2, s3)
    isems = (i0, i1)

    @pl.loop(0, 40)
    def _(i):
        for k in range(8):
            zb[i, pl.ds(k * 16, 16)] = z16

    node0 = s * NODE_ROWS_PER_TILE
    zds = []
    for t in range(NODE_ROWS_PER_TILE // 40):
        zds.append(pltpu.async_copy(zb, acc.at[pl.ds(node0 + t * 40, 40)],
                                    gsems[t % 4]))
    for dz in zds:
        dz.wait()
    plsc.subcore_barrier()

    base = w * UNITS_PER_TILE

    def fetch_idx(g, b):
        r0 = jnp.minimum(base + g * GUNITS, ERU - GUNITS)
        return (pltpu.async_copy(srcr.at[pl.ds(r0, GUNITS)], idx.at[b, 0],
                                 isems[b]),
                pltpu.async_copy(dstr.at[pl.ds(r0, GUNITS)], idx.at[b, 1],
                                 isems[b]))

    for d0 in fetch_idx(0, 0):
        d0.wait()

    def process_group(g, b):
        nxt = fetch_idx(g + 1, 1 - b)
        gd = [None] * GUNITS
        sd = [None] * GUNITS
        for u in range(GUNITS):
            sl = u % NSLOT
            if u >= NSLOT:
                sd[u - NSLOT].wait()
            gd[u] = pltpu.async_copy(h.at[idx.at[b, 0, u]], rows.at[sl],
                                     gsems[sl])
            if u >= 2:
                gd[u - 2].wait()
                sd[u - 2] = pltpu.async_copy(rows.at[(u - 2) % NSLOT],
                                             acc.at[idx.at[b, 1, u - 2]],
                                             ssems[(u - 2) % NSLOT], add=True)
        for u in (GUNITS - 2, GUNITS - 1):
            gd[u].wait()
            sd[u] = pltpu.async_copy(rows.at[u % NSLOT],
                                     acc.at[idx.at[b, 1, u]],
                                     ssems[u % NSLOT], add=True)
        for u in range(GUNITS - NSLOT, GUNITS):
            sd[u].wait()
        for d1 in nxt:
            d1.wait()

    @pl.loop(0, SEG_GROUPS // 2)
    def _(t):
        process_group(2 * t, 0)
        process_group(2 * t + 1, 1)

    plsc.subcore_barrier()
    for t in range(NODE_ROWS_PER_TILE // 128):
        sl = pl.ds(node0 + t * 128, 128)
        pltpu.sync_copy(acc.at[sl], out.at[c, sl])


def _make_seg_kernel():
    return pl.kernel(
        _seg_body,
        out_type=jax.ShapeDtypeStruct((NC, NP, D), jnp.float32),
        mesh=_mesh(),
        compiler_params=_sc_compiler_params(),
        scratch_types=[
            pltpu.VMEM_SHARED((NP, D), jnp.float32),
            pltpu.VMEM((NSLOT, UNIT, D), jnp.float32),
            pltpu.VMEM((2, 2, GUNITS, UNIT), jnp.int32),
            pltpu.VMEM((40, D), jnp.float32),
        ] + [pltpu.SemaphoreType.DMA] * 10,
    )


def _scales_body(cnt_ref, x_ref, so_ref, si_ref, h1_ref):
    cnt = cnt_ref[...]
    deg = jnp.maximum(jnp.sum(cnt, axis=1), 1.0)
    rt = jnp.transpose(lax.rsqrt(deg))
    ones_row = jnp.ones((1, D), jnp.float32)
    so = rt[:, 0:1] * ones_row
    so_ref[...] = so
    si_ref[...] = rt[:, 1:2] * ones_row
    h1_ref[...] = x_ref[...] * so


_scales_call = pl.pallas_call(
    _scales_body,
    grid=(GRID,),
    in_specs=[pl.BlockSpec((2, NW, NBLK), lambda i: (0, 0, i)),
              pl.BlockSpec((NBLK, D), lambda i: (i, 0))],
    out_specs=[pl.BlockSpec((NBLK, D), lambda i: (i, 0)),
               pl.BlockSpec((NBLK, D), lambda i: (i, 0)),
               pl.BlockSpec((NBLK, D), lambda i: (i, 0))],
    out_shape=[jax.ShapeDtypeStruct((NP, D), jnp.float32),
               jax.ShapeDtypeStruct((NP, D), jnp.float32),
               jax.ShapeDtypeStruct((NP, D), jnp.float32)],
)


def _combine1_body(parts_ref, si_ref, so_ref, w_ref, b_ref, o_ref):
    pp = parts_ref[...]
    p = (pp[0] + pp[1]) * si_ref[...]
    z = lax.dot_general(p, w_ref[...], (((1,), (0,)), ((), ())),
                        preferred_element_type=jnp.float32,
                        precision=lax.Precision.HIGHEST)
    z = z + b_ref[...]
    o_ref[...] = jnp.maximum(z, 0.0) * so_ref[...]


def _combine2_body(parts_ref, si_ref, w_ref, b_ref, o_ref):
    pp = parts_ref[...]
    p = (pp[0] + pp[1]) * si_ref[...]
    z = lax.dot_general(p, w_ref[...], (((1,), (0,)), ((), ())),
                        preferred_element_type=jnp.float32,
                        precision=lax.Precision.HIGHEST)
    o_ref[...] = z + b_ref[...]


_nd_spec = pl.BlockSpec((NBLK, D), lambda i: (i, 0))

_combine1_call = pl.pallas_call(
    _combine1_body,
    grid=(GRID,),
    in_specs=[
        pl.BlockSpec((NC, NBLK, D), lambda i: (0, i, 0)),
        _nd_spec,
        _nd_spec,
        pl.BlockSpec((D, D), lambda i: (0, 0)),
        pl.BlockSpec((1, D), lambda i: (0, 0)),
    ],
    out_specs=_nd_spec,
    out_shape=jax.ShapeDtypeStruct((NP, D), jnp.float32),
)

NBLK2 = 400
_combine2_call = pl.pallas_call(
    _combine2_body,
    grid=(N // NBLK2,),
    in_specs=[
        pl.BlockSpec((NC, NBLK2, D), lambda i: (0, i, 0)),
        pl.BlockSpec((NBLK2, D), lambda i: (i, 0)),
        pl.BlockSpec((D, D), lambda i: (0, 0)),
        pl.BlockSpec((1, D), lambda i: (0, 0)),
    ],
    out_specs=pl.BlockSpec((NBLK2, D), lambda i: (i, 0)),
    out_shape=jax.ShapeDtypeStruct((N, D), jnp.float32),
)


def kernel(x, edge_index, W1, b1, W2, b2):
    src = edge_index[0]
    dst = edge_index[1]
    padlen = ER * 128 - E
    padidx = (N + (jnp.arange(padlen, dtype=jnp.int32) % (NP - N))
              ).astype(jnp.int32)
    srcr = jnp.concatenate([src, padidx]).reshape(ERU, UNIT)
    dstr = jnp.concatenate([dst, padidx]).reshape(ERU, UNIT)
    x_pad = jnp.pad(x, ((0, NP - N), (0, 0)))
    b1r = b1.reshape(1, D)
    b2r = b2.reshape(1, D)

    cnt = _make_deg_kernel()(srcr, dstr)
    so, si, h1 = _scales_call(cnt, x_pad)
    seg = _make_seg_kernel()
    parts1 = seg(h1, srcr, dstr)
    h2 = _combine1_call(parts1, si, so, W1, b1r)
    parts2 = seg(h2, srcr, dstr)
    return _combine2_call(parts2, si, W2, b2r)


# speedup vs baseline: 11.6475x; 1.0063x over previous
import dataclasses
import functools

import jax
import jax.numpy as jnp
from jax import lax
from jax.experimental import pallas as pl
from jax.experimental.pallas import tpu as pltpu
from jax.experimental.pallas import tpu_sc as plsc

N = 10000
NP = 10240
D = 128
E = 320000
ER = 2560
NC, NS = 2, 16
NW = NC * NS
ROWS_PER_TILE = ER // NW
GROUPS = ROWS_PER_TILE // 8
NODE_ROWS_PER_TILE = NP // NS
NBLK = 512
GRID = NP // NBLK


def _mesh():
    return plsc.VectorSubcoreMesh(core_axis_name="c", subcore_axis_name="s")


def _sc_compiler_params():
    cp = pltpu.CompilerParams()
    if "needs_layout_passes" in pltpu.CompilerParams.__dataclass_fields__:
        cp = dataclasses.replace(cp, needs_layout_passes=False)
    return cp


DEG_GROUP = 8
DEG_NGRP = (E // 64) // DEG_GROUP
DEG_ITERS = -(-DEG_NGRP // NW)


def _deg_body(er, out, acc_s, acc_d, idx, i0, i1):
    c = lax.axis_index("c")
    s = lax.axis_index("s")
    w = c * NS + s
    z16 = jnp.zeros((16,), jnp.float32)
    isems = (i0, i1)

    @pl.loop(0, NP // 16)
    def _(i):
        acc_s[pl.ds(i * 16, 16)] = z16
        acc_d[pl.ds(i * 16, 16)] = z16

    ones = jnp.ones((16,), jnp.float32)

    def fetch(h, i, b):
        g = jnp.minimum(w + NW * i, DEG_NGRP - 1)
        return pltpu.async_copy(er.at[h, pl.ds(g * DEG_GROUP, DEG_GROUP)],
                                idx.at[b], isems[b])

    for h, acc in ((0, acc_s), (1, acc_d)):
        fetch(h, 0, 0).wait()

        def body(i, b, acc=acc, h=h):
            nxt = fetch(h, i + 1, 1 - b)

            @pl.when(w + NW * i < DEG_NGRP)
            def _():
                for j in range(DEG_GROUP):
                    for k in range(4):
                        iv = idx[b, j, pl.ds(k * 16, 16)]
                        plsc.addupdate_scatter(acc, [iv], ones)

            nxt.wait()

        @pl.loop(0, DEG_ITERS // 2)
        def _(t):
            body(2 * t, 0)
            body(2 * t + 1, 1)

    pltpu.sync_copy(acc_s, out.at[0, w])
    pltpu.sync_copy(acc_d, out.at[1, w])


def _make_deg_kernel():
    return pl.kernel(
        _deg_body,
        out_type=jax.ShapeDtypeStruct((2, NW, NP), jnp.float32),
        mesh=_mesh(),
        compiler_params=_sc_compiler_params(),
        scratch_types=[
            pltpu.VMEM((NP,), jnp.float32),
            pltpu.VMEM((NP,), jnp.float32),
            pltpu.VMEM((2, DEG_GROUP, 64), jnp.int32),
            pltpu.SemaphoreType.DMA,
            pltpu.SemaphoreType.DMA,
        ],
    )


UNIT = 64
UNITS_PER_TILE = ROWS_PER_TILE * 128 // UNIT
GUNITS = 16
SEG_GROUPS = UNITS_PER_TILE // GUNITS
ERU = ER * 128 // UNIT
NSLOT = 4


def _seg_body(h, srcr, dstr, out, acc, rows, idx, zb,
              g0, g1, g2, g3, s0, s1, s2, s3, i0, i1):
    c = lax.axis_index("c")
    s = lax.axis_index("s")
    w = c * NS + s
    z16 = jnp.zeros((16,), jnp.float32)
    gsems = (g0, g1, g2, g3)
    ssems = (s0, s1, s2, s3)
    isems = (i0, i1)

    @pl.loop(0, 40)
    def _(i):
        for k in range(8):
            zb[i, pl.ds(k * 16, 16)] = z16

    node0 = s * NODE_ROWS_PER_TILE
    zds = []
    for t in range(NODE_ROWS_PER_TILE // 40):
        zds.append(pltpu.async_copy(zb, acc.at[pl.ds(node0 + t * 40, 40)],
                                    gsems[t % 4]))
    for dz in zds:
        dz.wait()
    plsc.subcore_barrier()

    base = w * UNITS_PER_TILE

    def fetch_idx(g, b):
        r0 = jnp.minimum(base + g * GUNITS, ERU - GUNITS)
        return (pltpu.async_copy(srcr.at[pl.ds(r0, GUNITS)], idx.at[b, 0],
                                 isems[b]),
                pltpu.async_copy(dstr.at[pl.ds(r0, GUNITS)], idx.at[b, 1],
                                 isems[b]))

    for d0 in fetch_idx(0, 0):
        d0.wait()

    def process_group(g, b):
        nxt = fetch_idx(g + 1, 1 - b)
        gd = [None] * GUNITS
        sd = [None] * GUNITS
        for u in range(GUNITS):
            sl = u % NSLOT
            if u >= NSLOT:
                sd[u - NSLOT].wait()
            gd[u] = pltpu.async_copy(h.at[idx.at[b, 0, u]], rows.at[sl],
                                     gsems[sl])
            if u >= 2:
                gd[u - 2].wait()
                sd[u - 2] = pltpu.async_copy(rows.at[(u - 2) % NSLOT],
                                             acc.at[idx.at[b, 1, u - 2]],
                                             ssems[(u - 2) % NSLOT], add=True)
        for u in (GUNITS - 2, GUNITS - 1):
            gd[u].wait()
            sd[u] = pltpu.async_copy(rows.at[u % NSLOT],
                                     acc.at[idx.at[b, 1, u]],
                                     ssems[u % NSLOT], add=True)
        for u in range(GUNITS - NSLOT, GUNITS):
            sd[u].wait()
        for d1 in nxt:
            d1.wait()

    @pl.loop(0, SEG_GROUPS // 2)
    def _(t):
        process_group(2 * t, 0)
        process_group(2 * t + 1, 1)

    plsc.subcore_barrier()
    for t in range(NODE_ROWS_PER_TILE // 128):
        sl = pl.ds(node0 + t * 128, 128)
        pltpu.sync_copy(acc.at[sl], out.at[c, sl])


def _make_seg_kernel():
    return pl.kernel(
        _seg_body,
        out_type=jax.ShapeDtypeStruct((NC, NP, D), jnp.float32),
        mesh=_mesh(),
        compiler_params=_sc_compiler_params(),
        scratch_types=[
            pltpu.VMEM_SHARED((NP, D), jnp.float32),
            pltpu.VMEM((NSLOT, UNIT, D), jnp.float32),
            pltpu.VMEM((2, 2, GUNITS, UNIT), jnp.int32),
            pltpu.VMEM((40, D), jnp.float32),
        ] + [pltpu.SemaphoreType.DMA] * 10,
    )


def _scales_body(cnt_ref, x_ref, so_ref, si_ref, h1_ref):
    cnt = cnt_ref[...]
    deg = jnp.maximum(jnp.sum(cnt, axis=1), 1.0)
    rt = jnp.transpose(lax.rsqrt(deg))
    ones_row = jnp.ones((1, D), jnp.float32)
    so = rt[:, 0:1] * ones_row
    so_ref[...] = so
    si_ref[...] = rt[:, 1:2] * ones_row
    h1_ref[...] = x_ref[...] * so


_scales_call = pl.pallas_call(
    _scales_body,
    grid=(GRID,),
    in_specs=[pl.BlockSpec((2, NW, NBLK), lambda i: (0, 0, i)),
              pl.BlockSpec((NBLK, D), lambda i: (i, 0))],
    out_specs=[pl.BlockSpec((NBLK, D), lambda i: (i, 0)),
               pl.BlockSpec((NBLK, D), lambda i: (i, 0)),
               pl.BlockSpec((NBLK, D), lambda i: (i, 0))],
    out_shape=[jax.ShapeDtypeStruct((NP, D), jnp.float32),
               jax.ShapeDtypeStruct((NP, D), jnp.float32),
               jax.ShapeDtypeStruct((NP, D), jnp.float32)],
)


def _combine1_body(parts_ref, si_ref, so_ref, w_ref, b_ref, o_ref):
    pp = parts_ref[...]
    p = (pp[0] + pp[1]) * si_ref[...]
    z = lax.dot_general(p, w_ref[...], (((1,), (0,)), ((), ())),
                        preferred_element_type=jnp.float32,
                        precision=lax.Precision.HIGHEST)
    z = z + b_ref[...]
    o_ref[...] = jnp.maximum(z, 0.0) * so_ref[...]


def _combine2_body(parts_ref, si_ref, w_ref, b_ref, o_ref):
    pp = parts_ref[...]
    p = (pp[0] + pp[1]) * si_ref[...]
    z = lax.dot_general(p, w_ref[...], (((1,), (0,)), ((), ())),
                        preferred_element_type=jnp.float32,
                        precision=lax.Precision.HIGHEST)
    o_ref[...] = z + b_ref[...]


_nd_spec = pl.BlockSpec((NBLK, D), lambda i: (i, 0))

_combine1_call = pl.pallas_call(
    _combine1_body,
    grid=(GRID,),
    in_specs=[
        pl.BlockSpec((NC, NBLK, D), lambda i: (0, i, 0)),
        _nd_spec,
        _nd_spec,
        pl.BlockSpec((D, D), lambda i: (0, 0)),
        pl.BlockSpec((1, D), lambda i: (0, 0)),
    ],
    out_specs=_nd_spec,
    out_shape=jax.ShapeDtypeStruct((NP, D), jnp.float32),
)

NBLK2 = 400
_combine2_call = pl.pallas_call(
    _combine2_body,
    grid=(N // NBLK2,),
    in_specs=[
        pl.BlockSpec((NC, NBLK2, D), lambda i: (0, i, 0)),
        pl.BlockSpec((NBLK2, D), lambda i: (i, 0)),
        pl.BlockSpec((D, D), lambda i: (0, 0)),
        pl.BlockSpec((1, D), lambda i: (0, 0)),
    ],
    out_specs=pl.BlockSpec((NBLK2, D), lambda i: (i, 0)),
    out_shape=jax.ShapeDtypeStruct((N, D), jnp.float32),
)


def kernel(x, edge_index, W1, b1, W2, b2):
    src = edge_index[0]
    dst = edge_index[1]
    padlen = ER * 128 - E
    padidx = (N + (jnp.arange(padlen, dtype=jnp.int32) % (NP - N))
              ).astype(jnp.int32)
    srcr = jnp.concatenate([src, padidx]).reshape(ERU, UNIT)
    dstr = jnp.concatenate([dst, padidx]).reshape(ERU, UNIT)
    x_pad = jnp.pad(x, ((0, NP - N), (0, 0)))
    b1r = b1.reshape(1, D)
    b2r = b2.reshape(1, D)

    er = edge_index.reshape(2, E // 64, 64)
    cnt = _make_deg_kernel()(er)
    so, si, h1 = _scales_call(cnt, x_pad)
    seg = _make_seg_kernel()
    parts1 = seg(h1, srcr, dstr)
    h2 = _combine1_call(parts1, si, so, W1, b1r)
    parts2 = seg(h2, srcr, dstr)
    return _combine2_call(parts2, si, W2, b2r)


# speedup vs baseline: 12.2225x; 1.0494x over previous
import dataclasses
import functools

import jax
import jax.numpy as jnp
from jax import lax
from jax.experimental import pallas as pl
from jax.experimental.pallas import tpu as pltpu
from jax.experimental.pallas import tpu_sc as plsc

N = 10000
NP = 10240
D = 128
E = 320000
ER = 2560
NC, NS = 2, 16
NW = NC * NS
ROWS_PER_TILE = ER // NW
GROUPS = ROWS_PER_TILE // 8
NODE_ROWS_PER_TILE = NP // NS
NBLK = 512
GRID = NP // NBLK


def _mesh():
    return plsc.VectorSubcoreMesh(core_axis_name="c", subcore_axis_name="s")


def _sc_compiler_params():
    cp = pltpu.CompilerParams()
    if "needs_layout_passes" in pltpu.CompilerParams.__dataclass_fields__:
        cp = dataclasses.replace(cp, needs_layout_passes=False)
    return cp


DEG_CHUNK = 2560
DEG_NGRP = E // DEG_CHUNK
DEG_ITERS = -(-DEG_NGRP // NW)


def _deg_body(ei, out, acc_s, acc_d, idx, i0, i1):
    c = lax.axis_index("c")
    s = lax.axis_index("s")
    w = c * NS + s
    z16 = jnp.zeros((16,), jnp.float32)
    isems = (i0, i1)

    @pl.loop(0, NP // 16)
    def _(i):
        acc_s[pl.ds(i * 16, 16)] = z16
        acc_d[pl.ds(i * 16, 16)] = z16

    ones = jnp.ones((16,), jnp.float32)

    def fetch(h, i, b):
        g = jnp.minimum(w + NW * i, DEG_NGRP - 1)
        return pltpu.async_copy(ei.at[h, pl.ds(g * DEG_CHUNK, DEG_CHUNK)],
                                idx.at[b], isems[b])

    for h, acc in ((0, acc_s), (1, acc_d)):
        fetch(h, 0, 0).wait()

        def body(i, b, acc=acc, h=h):
            nxt = fetch(h, i + 1, 1 - b)

            @pl.when(w + NW * i < DEG_NGRP)
            def _():
                @pl.loop(0, DEG_CHUNK // 16)
                def _(n):
                    iv = idx[b, pl.ds(n * 16, 16)]
                    plsc.addupdate_scatter(acc, [iv], ones)

            nxt.wait()

        @pl.loop(0, DEG_ITERS // 2)
        def _(t):
            body(2 * t, 0)
            body(2 * t + 1, 1)

    pltpu.sync_copy(acc_s, out.at[0, w])
    pltpu.sync_copy(acc_d, out.at[1, w])


def _make_deg_kernel():
    return pl.kernel(
        _deg_body,
        out_type=jax.ShapeDtypeStruct((2, NW, NP), jnp.float32),
        mesh=_mesh(),
        compiler_params=_sc_compiler_params(),
        scratch_types=[
            pltpu.VMEM((NP,), jnp.float32),
            pltpu.VMEM((NP,), jnp.float32),
            pltpu.VMEM((2, DEG_CHUNK), jnp.int32),
            pltpu.SemaphoreType.DMA,
            pltpu.SemaphoreType.DMA,
        ],
    )


UNIT = 64
UNITS_PER_TILE = ROWS_PER_TILE * 128 // UNIT
GUNITS = 16
SEG_GROUPS = UNITS_PER_TILE // GUNITS
ERU = ER * 128 // UNIT
NSLOT = 4


def _seg_body(h, srcr, dstr, out, acc, rows, idx, zb,
              g0, g1, g2, g3, s0, s1, s2, s3, i0, i1):
    c = lax.axis_index("c")
    s = lax.axis_index("s")
    w = c * NS + s
    z16 = jnp.zeros((16,), jnp.float32)
    gsems = (g0, g1, g2, g3)
    ssems = (s0, s1, s2, s3)
    isems = (i0, i1)

    @pl.loop(0, 40)
    def _(i):
        for k in range(8):
            zb[i, pl.ds(k * 16, 16)] = z16

    node0 = s * NODE_ROWS_PER_TILE
    zds = []
    for t in range(NODE_ROWS_PER_TILE // 40):
        zds.append(pltpu.async_copy(zb, acc.at[pl.ds(node0 + t * 40, 40)],
                                    gsems[t % 4]))
    for dz in zds:
        dz.wait()
    plsc.subcore_barrier()

    base = w * UNITS_PER_TILE

    def fetch_idx(g, b):
        r0 = jnp.minimum(base + g * GUNITS, ERU - GUNITS)
        return (pltpu.async_copy(srcr.at[pl.ds(r0, GUNITS)], idx.at[b, 0],
                                 isems[b]),
                pltpu.async_copy(dstr.at[pl.ds(r0, GUNITS)], idx.at[b, 1],
                                 isems[b]))

    for d0 in fetch_idx(0, 0):
        d0.wait()

    def process_group(g, b):
        nxt = fetch_idx(g + 1, 1 - b)
        gd = [None] * GUNITS
        sd = [None] * GUNITS
        for u in range(GUNITS):
            sl = u % NSLOT
            if u >= NSLOT:
                sd[u - NSLOT].wait()
            gd[u] = pltpu.async_copy(h.at[idx.at[b, 0, u]], rows.at[sl],
                                     gsems[sl])
            if u >= 2:
                gd[u - 2].wait()
                sd[u - 2] = pltpu.async_copy(rows.at[(u - 2) % NSLOT],
                                             acc.at[idx.at[b, 1, u - 2]],
                                             ssems[(u - 2) % NSLOT], add=True)
        for u in (GUNITS - 2, GUNITS - 1):
            gd[u].wait()
            sd[u] = pltpu.async_copy(rows.at[u % NSLOT],
                                     acc.at[idx.at[b, 1, u]],
                                     ssems[u % NSLOT], add=True)
        for u in range(GUNITS - NSLOT, GUNITS):
            sd[u].wait()
        for d1 in nxt:
            d1.wait()

    @pl.loop(0, SEG_GROUPS // 2)
    def _(t):
        process_group(2 * t, 0)
        process_group(2 * t + 1, 1)

    plsc.subcore_barrier()
    rds = []
    for t in range(NODE_ROWS_PER_TILE // 128):
        sl = pl.ds(node0 + t * 128, 128)
        rds.append(pltpu.async_copy(acc.at[sl], out.at[c, sl], gsems[t % 4]))
    for r in rds:
        r.wait()


def _make_seg_kernel():
    return pl.kernel(
        _seg_body,
        out_type=jax.ShapeDtypeStruct((NC, NP, D), jnp.float32),
        mesh=_mesh(),
        compiler_params=_sc_compiler_params(),
        scratch_types=[
            pltpu.VMEM_SHARED((NP, D), jnp.float32),
            pltpu.VMEM((NSLOT, UNIT, D), jnp.float32),
            pltpu.VMEM((2, 2, GUNITS, UNIT), jnp.int32),
            pltpu.VMEM((40, D), jnp.float32),
        ] + [pltpu.SemaphoreType.DMA] * 10,
    )


def _scales_body(cnt_ref, x_ref, so_ref, si_ref, h1_ref):
    cnt = cnt_ref[...]
    deg = jnp.maximum(jnp.sum(cnt, axis=1), 1.0)
    rt = jnp.transpose(lax.rsqrt(deg))
    ones_row = jnp.ones((1, D), jnp.float32)
    so = rt[:, 0:1] * ones_row
    so_ref[...] = so
    si_ref[...] = rt[:, 1:2] * ones_row
    h1_ref[...] = x_ref[...] * so


_scales_call = pl.pallas_call(
    _scales_body,
    grid=(GRID,),
    in_specs=[pl.BlockSpec((2, NW, NBLK), lambda i: (0, 0, i)),
              pl.BlockSpec((NBLK, D), lambda i: (i, 0))],
    out_specs=[pl.BlockSpec((NBLK, D), lambda i: (i, 0)),
               pl.BlockSpec((NBLK, D), lambda i: (i, 0)),
               pl.BlockSpec((NBLK, D), lambda i: (i, 0))],
    out_shape=[jax.ShapeDtypeStruct((NP, D), jnp.float32),
               jax.ShapeDtypeStruct((NP, D), jnp.float32),
               jax.ShapeDtypeStruct((NP, D), jnp.float32)],
)


def _combine1_body(parts_ref, si_ref, so_ref, w_ref, b_ref, o_ref):
    pp = parts_ref[...]
    p = (pp[0] + pp[1]) * si_ref[...]
    z = lax.dot_general(p, w_ref[...], (((1,), (0,)), ((), ())),
                        preferred_element_type=jnp.float32,
                        precision=lax.Precision.HIGHEST)
    z = z + b_ref[...]
    o_ref[...] = jnp.maximum(z, 0.0) * so_ref[...]


def _combine2_body(parts_ref, si_ref, w_ref, b_ref, o_ref):
    pp = parts_ref[...]
    p = (pp[0] + pp[1]) * si_ref[...]
    z = lax.dot_general(p, w_ref[...], (((1,), (0,)), ((), ())),
                        preferred_element_type=jnp.float32,
                        precision=lax.Precision.HIGHEST)
    o_ref[...] = z + b_ref[...]


_nd_spec = pl.BlockSpec((NBLK, D), lambda i: (i, 0))

_combine1_call = pl.pallas_call(
    _combine1_body,
    grid=(GRID,),
    in_specs=[
        pl.BlockSpec((NC, NBLK, D), lambda i: (0, i, 0)),
        _nd_spec,
        _nd_spec,
        pl.BlockSpec((D, D), lambda i: (0, 0)),
        pl.BlockSpec((1, D), lambda i: (0, 0)),
    ],
    out_specs=_nd_spec,
    out_shape=jax.ShapeDtypeStruct((NP, D), jnp.float32),
)

NBLK2 = 400
_combine2_call = pl.pallas_call(
    _combine2_body,
    grid=(N // NBLK2,),
    in_specs=[
        pl.BlockSpec((NC, NBLK2, D), lambda i: (0, i, 0)),
        pl.BlockSpec((NBLK2, D), lambda i: (i, 0)),
        pl.BlockSpec((D, D), lambda i: (0, 0)),
        pl.BlockSpec((1, D), lambda i: (0, 0)),
    ],
    out_specs=pl.BlockSpec((NBLK2, D), lambda i: (i, 0)),
    out_shape=jax.ShapeDtypeStruct((N, D), jnp.float32),
)


def kernel(x, edge_index, W1, b1, W2, b2):
    src = edge_index[0]
    dst = edge_index[1]
    padlen = ER * 128 - E
    padidx = (N + (jnp.arange(padlen, dtype=jnp.int32) % (NP - N))
              ).astype(jnp.int32)
    srcr = jnp.concatenate([src, padidx]).reshape(ERU, UNIT)
    dstr = jnp.concatenate([dst, padidx]).reshape(ERU, UNIT)
    x_pad = jnp.pad(x, ((0, NP - N), (0, 0)))
    b1r = b1.reshape(1, D)
    b2r = b2.reshape(1, D)

    cnt = _make_deg_kernel()(edge_index)
    so, si, h1 = _scales_call(cnt, x_pad)
    seg = _make_seg_kernel()
    parts1 = seg(h1, srcr, dstr)
    h2 = _combine1_call(parts1, si, so, W1, b1r)
    parts2 = seg(h2, srcr, dstr)
    return _combine2_call(parts2, si, W2, b2r)


# speedup vs baseline: 12.3906x; 1.0138x over previous
import dataclasses
import functools

import jax
import jax.numpy as jnp
from jax import lax
from jax.experimental import pallas as pl
from jax.experimental.pallas import tpu as pltpu
from jax.experimental.pallas import tpu_sc as plsc

N = 10000
NP = 10240
D = 128
E = 320000
ER = 2560
NC, NS = 2, 16
NW = NC * NS
ROWS_PER_TILE = ER // NW
GROUPS = ROWS_PER_TILE // 8
NODE_ROWS_PER_TILE = NP // NS
NBLK = 512
GRID = NP // NBLK


def _mesh():
    return plsc.VectorSubcoreMesh(core_axis_name="c", subcore_axis_name="s")


def _sc_compiler_params():
    cp = pltpu.CompilerParams()
    if "needs_layout_passes" in pltpu.CompilerParams.__dataclass_fields__:
        cp = dataclasses.replace(cp, needs_layout_passes=False)
    return cp


DEG_CHUNK = 2560
DEG_NGRP = E // DEG_CHUNK
DEG_ITERS = -(-DEG_NGRP // NW)


def _deg_body(ei, out, acc_s, acc_d, idx, i0, i1):
    c = lax.axis_index("c")
    s = lax.axis_index("s")
    w = c * NS + s
    z16 = jnp.zeros((16,), jnp.float32)
    isems = (i0, i1)

    @pl.loop(0, NP // 16)
    def _(i):
        acc_s[pl.ds(i * 16, 16)] = z16
        acc_d[pl.ds(i * 16, 16)] = z16

    ones = jnp.ones((16,), jnp.float32)

    def fetch(h, i, b):
        g = jnp.minimum(w + NW * i, DEG_NGRP - 1)
        return pltpu.async_copy(ei.at[h, pl.ds(g * DEG_CHUNK, DEG_CHUNK)],
                                idx.at[b], isems[b])

    for h, acc in ((0, acc_s), (1, acc_d)):
        fetch(h, 0, 0).wait()

        def body(i, b, acc=acc, h=h):
            nxt = fetch(h, i + 1, 1 - b)

            @pl.when(w + NW * i < DEG_NGRP)
            def _():
                @pl.loop(0, DEG_CHUNK // 16)
                def _(n):
                    iv = idx[b, pl.ds(n * 16, 16)]
                    plsc.addupdate_scatter(acc, [iv], ones)

            nxt.wait()

        @pl.loop(0, DEG_ITERS // 2)
        def _(t):
            body(2 * t, 0)
            body(2 * t + 1, 1)

    pltpu.sync_copy(acc_s, out.at[0, w])
    pltpu.sync_copy(acc_d, out.at[1, w])


def _make_deg_kernel():
    return pl.kernel(
        _deg_body,
        out_type=jax.ShapeDtypeStruct((2, NW, NP), jnp.float32),
        mesh=_mesh(),
        compiler_params=_sc_compiler_params(),
        scratch_types=[
            pltpu.VMEM((NP,), jnp.float32),
            pltpu.VMEM((NP,), jnp.float32),
            pltpu.VMEM((2, DEG_CHUNK), jnp.int32),
            pltpu.SemaphoreType.DMA,
            pltpu.SemaphoreType.DMA,
        ],
    )


UNIT = 64
UNITS_PER_TILE = ROWS_PER_TILE * 128 // UNIT
GUNITS = 16
SEG_GROUPS = UNITS_PER_TILE // GUNITS
ERU = ER * 128 // UNIT
NSLOT = 4


def _seg_body(h, srcr, dstr, out, acc, rows, idx, zb,
              g0, g1, g2, g3, s0, s1, s2, s3, i0, i1):
    c = lax.axis_index("c")
    s = lax.axis_index("s")
    w = c * NS + s
    z16 = jnp.zeros((16,), jnp.float32)
    gsems = (g0, g1, g2, g3)
    ssems = (s0, s1, s2, s3)
    isems = (i0, i1)

    @pl.loop(0, 40)
    def _(i):
        for k in range(8):
            zb[i, pl.ds(k * 16, 16)] = z16

    node0 = s * NODE_ROWS_PER_TILE
    zds = []
    for t in range(NODE_ROWS_PER_TILE // 40):
        zds.append(pltpu.async_copy(zb, acc.at[pl.ds(node0 + t * 40, 40)],
                                    gsems[t % 4]))
    for dz in zds:
        dz.wait()
    plsc.subcore_barrier()

    base = w * UNITS_PER_TILE

    def fetch_idx(g, b):
        r0 = jnp.minimum(base + g * GUNITS, ERU - GUNITS)
        return (pltpu.async_copy(srcr.at[pl.ds(r0, GUNITS)], idx.at[b, 0],
                                 isems[b]),
                pltpu.async_copy(dstr.at[pl.ds(r0, GUNITS)], idx.at[b, 1],
                                 isems[b]))

    for d0 in fetch_idx(0, 0):
        d0.wait()

    def process_group(g, b):
        nxt = fetch_idx(g + 1, 1 - b)
        gd = [None] * GUNITS
        sd = [None] * GUNITS
        for u in range(GUNITS):
            sl = u % NSLOT
            if u >= NSLOT:
                sd[u - NSLOT].wait()
            gd[u] = pltpu.async_copy(h.at[idx.at[b, 0, u]], rows.at[sl],
                                     gsems[sl])
            if u >= 2:
                gd[u - 2].wait()
                sd[u - 2] = pltpu.async_copy(rows.at[(u - 2) % NSLOT],
                                             acc.at[idx.at[b, 1, u - 2]],
                                             ssems[(u - 2) % NSLOT], add=True)
        for u in (GUNITS - 2, GUNITS - 1):
            gd[u].wait()
            sd[u] = pltpu.async_copy(rows.at[u % NSLOT],
                                     acc.at[idx.at[b, 1, u]],
                                     ssems[u % NSLOT], add=True)
        for u in range(GUNITS - NSLOT, GUNITS):
            sd[u].wait()
        for d1 in nxt:
            d1.wait()

    @pl.loop(0, SEG_GROUPS // 2)
    def _(t):
        process_group(2 * t, 0)
        process_group(2 * t + 1, 1)

    plsc.subcore_barrier()
    rds = []
    for t in range(NODE_ROWS_PER_TILE // 128):
        sl = pl.ds(node0 + t * 128, 128)
        rds.append(pltpu.async_copy(acc.at[sl], out.at[c, sl], gsems[t % 4]))
    for r in rds:
        r.wait()


def _make_seg_kernel():
    return pl.kernel(
        _seg_body,
        out_type=jax.ShapeDtypeStruct((NC, NP, D), jnp.float32),
        mesh=_mesh(),
        compiler_params=_sc_compiler_params(),
        scratch_types=[
            pltpu.VMEM_SHARED((NP, D), jnp.float32),
            pltpu.VMEM((NSLOT, UNIT, D), jnp.float32),
            pltpu.VMEM((2, 2, GUNITS, UNIT), jnp.int32),
            pltpu.VMEM((40, D), jnp.float32),
        ] + [pltpu.SemaphoreType.DMA] * 10,
    )


def _scales_body(cnt_ref, x_ref, so_ref, si_ref, h1_ref):
    cnt = cnt_ref[...]
    deg = jnp.maximum(jnp.sum(cnt, axis=1), 1.0)
    rt = jnp.transpose(lax.rsqrt(deg))
    ones_row = jnp.ones((1, D), jnp.float32)
    so = rt[:, 0:1] * ones_row
    so_ref[...] = so
    si_ref[...] = rt[:, 1:2] * ones_row
    h1_ref[...] = x_ref[...] * so


_scales_call = pl.pallas_call(
    _scales_body,
    grid=(GRID,),
    in_specs=[pl.BlockSpec((2, NW, NBLK), lambda i: (0, 0, i)),
              pl.BlockSpec((NBLK, D), lambda i: (i, 0))],
    out_specs=[pl.BlockSpec((NBLK, D), lambda i: (i, 0)),
               pl.BlockSpec((NBLK, D), lambda i: (i, 0)),
               pl.BlockSpec((NBLK, D), lambda i: (i, 0))],
    out_shape=[jax.ShapeDtypeStruct((NP, D), jnp.float32),
               jax.ShapeDtypeStruct((NP, D), jnp.float32),
               jax.ShapeDtypeStruct((NP, D), jnp.float32)],
)


def _combine1_body(parts_ref, si_ref, so_ref, w_ref, b_ref, o_ref):
    pp = parts_ref[...]
    p = (pp[0] + pp[1]) * si_ref[...]
    z = lax.dot_general(p, w_ref[...], (((1,), (0,)), ((), ())),
                        preferred_element_type=jnp.float32,
                        precision=lax.Precision.DEFAULT)
    z = z + b_ref[...]
    o_ref[...] = jnp.maximum(z, 0.0) * so_ref[...]


def _combine2_body(parts_ref, si_ref, w_ref, b_ref, o_ref):
    pp = parts_ref[...]
    p = (pp[0] + pp[1]) * si_ref[...]
    z = lax.dot_general(p, w_ref[...], (((1,), (0,)), ((), ())),
                        preferred_element_type=jnp.float32,
                        precision=lax.Precision.DEFAULT)
    o_ref[...] = z + b_ref[...]


_nd_spec = pl.BlockSpec((NBLK, D), lambda i: (i, 0))

_combine1_call = pl.pallas_call(
    _combine1_body,
    grid=(GRID,),
    in_specs=[
        pl.BlockSpec((NC, NBLK, D), lambda i: (0, i, 0)),
        _nd_spec,
        _nd_spec,
        pl.BlockSpec((D, D), lambda i: (0, 0)),
        pl.BlockSpec((1, D), lambda i: (0, 0)),
    ],
    out_specs=_nd_spec,
    out_shape=jax.ShapeDtypeStruct((NP, D), jnp.float32),
)

NBLK2 = 400
_combine2_call = pl.pallas_call(
    _combine2_body,
    grid=(N // NBLK2,),
    in_specs=[
        pl.BlockSpec((NC, NBLK2, D), lambda i: (0, i, 0)),
        pl.BlockSpec((NBLK2, D), lambda i: (i, 0)),
        pl.BlockSpec((D, D), lambda i: (0, 0)),
        pl.BlockSpec((1, D), lambda i: (0, 0)),
    ],
    out_specs=pl.BlockSpec((NBLK2, D), lambda i: (i, 0)),
    out_shape=jax.ShapeDtypeStruct((N, D), jnp.float32),
)


def kernel(x, edge_index, W1, b1, W2, b2):
    src = edge_index[0]
    dst = edge_index[1]
    padlen = ER * 128 - E
    padidx = (N + (jnp.arange(padlen, dtype=jnp.int32) % (NP - N))
              ).astype(jnp.int32)
    srcr = jnp.concatenate([src, padidx]).reshape(ERU, UNIT)
    dstr = jnp.concatenate([dst, padidx]).reshape(ERU, UNIT)
    x_pad = jnp.pad(x, ((0, NP - N), (0, 0)))
    b1r = b1.reshape(1, D)
    b2r = b2.reshape(1, D)

    cnt = _make_deg_kernel()(edge_index)
    so, si, h1 = _scales_call(cnt, x_pad)
    seg = _make_seg_kernel()
    parts1 = seg(h1, srcr, dstr)
    h2 = _combine1_call(parts1, si, so, W1, b1r)
    parts2 = seg(h2, srcr, dstr)
    return _combine2_call(parts2, si, W2, b2r)


# speedup vs baseline: 12.5247x; 1.0108x over previous
import dataclasses
import functools

import jax
import jax.numpy as jnp
from jax import lax
from jax.experimental import pallas as pl
from jax.experimental.pallas import tpu as pltpu
from jax.experimental.pallas import tpu_sc as plsc

N = 10000
NP = 10240
D = 128
E = 320000
ER = 2560
NC, NS = 2, 16
NW = NC * NS
ROWS_PER_TILE = ER // NW
GROUPS = ROWS_PER_TILE // 8
NODE_ROWS_PER_TILE = NP // NS
NBLK = 512
GRID = NP // NBLK


def _mesh():
    return plsc.VectorSubcoreMesh(core_axis_name="c", subcore_axis_name="s")


def _sc_compiler_params():
    cp = pltpu.CompilerParams()
    if "needs_layout_passes" in pltpu.CompilerParams.__dataclass_fields__:
        cp = dataclasses.replace(cp, needs_layout_passes=False)
    return cp


DEG_CHUNK = 2560
DEG_NGRP = E // DEG_CHUNK
DEG_ITERS = -(-DEG_NGRP // NW)


def _deg_body(ei, out, acc_s, acc_d, idx, i0, i1):
    c = lax.axis_index("c")
    s = lax.axis_index("s")
    w = c * NS + s
    z16 = jnp.zeros((16,), jnp.float32)
    isems = (i0, i1)

    @pl.loop(0, NP // 16)
    def _(i):
        acc_s[pl.ds(i * 16, 16)] = z16
        acc_d[pl.ds(i * 16, 16)] = z16

    ones = jnp.ones((16,), jnp.float32)

    def fetch(h, i, b):
        g = jnp.minimum(w + NW * i, DEG_NGRP - 1)
        return pltpu.async_copy(ei.at[h, pl.ds(g * DEG_CHUNK, DEG_CHUNK)],
                                idx.at[b], isems[b])

    for h, acc in ((0, acc_s), (1, acc_d)):
        fetch(h, 0, 0).wait()

        def body(i, b, acc=acc, h=h):
            nxt = fetch(h, i + 1, 1 - b)

            @pl.when(w + NW * i < DEG_NGRP)
            def _():
                @pl.loop(0, DEG_CHUNK // 16)
                def _(n):
                    iv = idx[b, pl.ds(n * 16, 16)]
                    plsc.addupdate_scatter(acc, [iv], ones)

            nxt.wait()

        @pl.loop(0, DEG_ITERS // 2)
        def _(t):
            body(2 * t, 0)
            body(2 * t + 1, 1)

    pltpu.sync_copy(acc_s, out.at[0, w])
    pltpu.sync_copy(acc_d, out.at[1, w])


def _make_deg_kernel():
    return pl.kernel(
        _deg_body,
        out_type=jax.ShapeDtypeStruct((2, NW, NP), jnp.float32),
        mesh=_mesh(),
        compiler_params=_sc_compiler_params(),
        scratch_types=[
            pltpu.VMEM((NP,), jnp.float32),
            pltpu.VMEM((NP,), jnp.float32),
            pltpu.VMEM((2, DEG_CHUNK), jnp.int32),
            pltpu.SemaphoreType.DMA,
            pltpu.SemaphoreType.DMA,
        ],
    )


UNIT = 64
UNITS_PER_TILE = ROWS_PER_TILE * 128 // UNIT
GUNITS = 16
SEG_GROUPS = UNITS_PER_TILE // GUNITS
ERU = ER * 128 // UNIT
NSLOT = 4


def _seg_body(h, srcr, dstr, out, acc, rows, idx, zb,
              g0, g1, g2, g3, s0, s1, s2, s3, i0, i1):
    c = lax.axis_index("c")
    s = lax.axis_index("s")
    w = c * NS + s
    z16 = jnp.zeros((16,), jnp.float32)
    gsems = (g0, g1, g2, g3)
    ssems = (s0, s1, s2, s3)
    isems = (i0, i1)

    @pl.loop(0, 40)
    def _(i):
        for k in range(8):
            zb[i, pl.ds(k * 16, 16)] = z16

    node0 = s * NODE_ROWS_PER_TILE
    zds = []
    for t in range(NODE_ROWS_PER_TILE // 40):
        zds.append(pltpu.async_copy(zb, acc.at[pl.ds(node0 + t * 40, 40)],
                                    gsems[t % 4]))
    for dz in zds:
        dz.wait()
    plsc.subcore_barrier()

    base = w * UNITS_PER_TILE

    def fetch_idx(g, b):
        r0 = jnp.minimum(base + g * GUNITS, ERU - GUNITS)
        return (pltpu.async_copy(srcr.at[pl.ds(r0, GUNITS)], idx.at[b, 0],
                                 isems[b]),
                pltpu.async_copy(dstr.at[pl.ds(r0, GUNITS)], idx.at[b, 1],
                                 isems[b]))

    for d0 in fetch_idx(0, 0):
        d0.wait()

    def process_group(g, b, prev_tail):
        nxt = fetch_idx(g + 1, 1 - b)
        gd = [None] * GUNITS
        sd = [None] * GUNITS
        for u in range(GUNITS):
            sl = u % NSLOT
            if u >= NSLOT:
                sd[u - NSLOT].wait()
            elif prev_tail is not None:
                prev_tail[u].wait()
            gd[u] = pltpu.async_copy(h.at[idx.at[b, 0, u]], rows.at[sl],
                                     gsems[sl])
            if u >= 2:
                gd[u - 2].wait()
                sd[u - 2] = pltpu.async_copy(rows.at[(u - 2) % NSLOT],
                                             acc.at[idx.at[b, 1, u - 2]],
                                             ssems[(u - 2) % NSLOT], add=True)
        for u in (GUNITS - 2, GUNITS - 1):
            gd[u].wait()
            sd[u] = pltpu.async_copy(rows.at[u % NSLOT],
                                     acc.at[idx.at[b, 1, u]],
                                     ssems[u % NSLOT], add=True)
        for d1 in nxt:
            d1.wait()
        return sd[GUNITS - NSLOT:]

    @pl.loop(0, SEG_GROUPS // 2)
    def _(t):
        tail0 = process_group(2 * t, 0, None)
        tail1 = process_group(2 * t + 1, 1, tail0)
        for d in tail1:
            d.wait()

    plsc.subcore_barrier()
    rds = []
    for t in range(NODE_ROWS_PER_TILE // 128):
        sl = pl.ds(node0 + t * 128, 128)
        rds.append(pltpu.async_copy(acc.at[sl], out.at[c, sl], gsems[t % 4]))
    for r in rds:
        r.wait()


def _make_seg_kernel():
    return pl.kernel(
        _seg_body,
        out_type=jax.ShapeDtypeStruct((NC, NP, D), jnp.float32),
        mesh=_mesh(),
        compiler_params=_sc_compiler_params(),
        scratch_types=[
            pltpu.VMEM_SHARED((NP, D), jnp.float32),
            pltpu.VMEM((NSLOT, UNIT, D), jnp.float32),
            pltpu.VMEM((2, 2, GUNITS, UNIT), jnp.int32),
            pltpu.VMEM((40, D), jnp.float32),
        ] + [pltpu.SemaphoreType.DMA] * 10,
    )


def _rt_scales(cnt):
    deg = jnp.maximum(jnp.sum(cnt, axis=1), 1.0)
    return jnp.transpose(lax.rsqrt(deg))


def _scales_body(cnt_ref, x_ref, si_ref, h1_ref):
    rt = _rt_scales(cnt_ref[...])
    ones_row = jnp.ones((1, D), jnp.float32)
    si_ref[...] = rt[:, 1:2] * ones_row
    h1_ref[...] = x_ref[...] * rt[:, 0:1]


_scales_call = pl.pallas_call(
    _scales_body,
    grid=(GRID,),
    in_specs=[pl.BlockSpec((2, NW, NBLK), lambda i: (0, 0, i)),
              pl.BlockSpec((NBLK, D), lambda i: (i, 0))],
    out_specs=[pl.BlockSpec((NBLK, D), lambda i: (i, 0)),
               pl.BlockSpec((NBLK, D), lambda i: (i, 0))],
    out_shape=[jax.ShapeDtypeStruct((NP, D), jnp.float32),
               jax.ShapeDtypeStruct((NP, D), jnp.float32)],
)


def _combine1_body(parts_ref, cnt_ref, w_ref, b_ref, o_ref):
    rt = _rt_scales(cnt_ref[...])
    pp = parts_ref[...]
    p = (pp[0] + pp[1]) * rt[:, 1:2]
    z = lax.dot_general(p, w_ref[...], (((1,), (0,)), ((), ())),
                        preferred_element_type=jnp.float32,
                        precision=lax.Precision.DEFAULT)
    z = z + b_ref[...]
    o_ref[...] = jnp.maximum(z, 0.0) * rt[:, 0:1]


def _combine2_body(parts_ref, si_ref, w_ref, b_ref, o_ref):
    pp = parts_ref[...]
    p = (pp[0] + pp[1]) * si_ref[...]
    z = lax.dot_general(p, w_ref[...], (((1,), (0,)), ((), ())),
                        preferred_element_type=jnp.float32,
                        precision=lax.Precision.DEFAULT)
    o_ref[...] = z + b_ref[...]


_nd_spec = pl.BlockSpec((NBLK, D), lambda i: (i, 0))

_combine1_call = pl.pallas_call(
    _combine1_body,
    grid=(GRID,),
    in_specs=[
        pl.BlockSpec((NC, NBLK, D), lambda i: (0, i, 0)),
        pl.BlockSpec((2, NW, NBLK), lambda i: (0, 0, i)),
        pl.BlockSpec((D, D), lambda i: (0, 0)),
        pl.BlockSpec((1, D), lambda i: (0, 0)),
    ],
    out_specs=_nd_spec,
    out_shape=jax.ShapeDtypeStruct((NP, D), jnp.float32),
)

NBLK2 = 400
_combine2_call = pl.pallas_call(
    _combine2_body,
    grid=(N // NBLK2,),
    in_specs=[
        pl.BlockSpec((NC, NBLK2, D), lambda i: (0, i, 0)),
        pl.BlockSpec((NBLK2, D), lambda i: (i, 0)),
        pl.BlockSpec((D, D), lambda i: (0, 0)),
        pl.BlockSpec((1, D), lambda i: (0, 0)),
    ],
    out_specs=pl.BlockSpec((NBLK2, D), lambda i: (i, 0)),
    out_shape=jax.ShapeDtypeStruct((N, D), jnp.float32),
)


def kernel(x, edge_index, W1, b1, W2, b2):
    src = edge_index[0]
    dst = edge_index[1]
    padlen = ER * 128 - E
    padidx = (N + (jnp.arange(padlen, dtype=jnp.int32) % (NP - N))
              ).astype(jnp.int32)
    srcr = jnp.concatenate([src, padidx]).reshape(ERU, UNIT)
    dstr = jnp.concatenate([dst, padidx]).reshape(ERU, UNIT)
    x_pad = jnp.pad(x, ((0, NP - N), (0, 0)))
    b1r = b1.reshape(1, D)
    b2r = b2.reshape(1, D)

    cnt = _make_deg_kernel()(edge_index)
    si, h1 = _scales_call(cnt, x_pad)
    seg = _make_seg_kernel()
    parts1 = seg(h1, srcr, dstr)
    h2 = _combine1_call(parts1, cnt, W1, b1r)
    parts2 = seg(h2, srcr, dstr)
    return _combine2_call(parts2, si, W2, b2r)


# speedup vs baseline: 13.2576x; 1.0585x over previous
import dataclasses
import functools

import jax
import jax.numpy as jnp
from jax import lax
from jax.experimental import pallas as pl
from jax.experimental.pallas import tpu as pltpu
from jax.experimental.pallas import tpu_sc as plsc

N = 10000
NP = 10240
D = 128
E = 320000
ER = 2560
NC, NS = 2, 16
NW = NC * NS
ROWS_PER_TILE = ER // NW
GROUPS = ROWS_PER_TILE // 8
NODE_ROWS_PER_TILE = NP // NS
NBLK = 1024
GRID = NP // NBLK


def _mesh():
    return plsc.VectorSubcoreMesh(core_axis_name="c", subcore_axis_name="s")


def _sc_compiler_params():
    cp = pltpu.CompilerParams()
    if "needs_layout_passes" in pltpu.CompilerParams.__dataclass_fields__:
        cp = dataclasses.replace(cp, needs_layout_passes=False)
    return cp


DEG_CHUNK = 2560
DEG_NGRP = E // DEG_CHUNK
DEG_ITERS = -(-DEG_NGRP // NW)


def _deg_body(ei, out, acc_s, acc_d, idx, i0, i1):
    c = lax.axis_index("c")
    s = lax.axis_index("s")
    w = c * NS + s
    z16 = jnp.zeros((16,), jnp.float32)
    isems = (i0, i1)

    @pl.loop(0, NP // 16)
    def _(i):
        acc_s[pl.ds(i * 16, 16)] = z16
        acc_d[pl.ds(i * 16, 16)] = z16

    ones = jnp.ones((16,), jnp.float32)

    def fetch(h, i, b):
        g = jnp.minimum(w + NW * i, DEG_NGRP - 1)
        return pltpu.async_copy(ei.at[h, pl.ds(g * DEG_CHUNK, DEG_CHUNK)],
                                idx.at[b], isems[b])

    for h, acc in ((0, acc_s), (1, acc_d)):
        fetch(h, 0, 0).wait()

        def body(i, b, acc=acc, h=h):
            nxt = fetch(h, i + 1, 1 - b)

            @pl.when(w + NW * i < DEG_NGRP)
            def _():
                @pl.loop(0, DEG_CHUNK // 16)
                def _(n):
                    iv = idx[b, pl.ds(n * 16, 16)]
                    plsc.addupdate_scatter(acc, [iv], ones)

            nxt.wait()

        @pl.loop(0, DEG_ITERS // 2)
        def _(t):
            body(2 * t, 0)
            body(2 * t + 1, 1)

    pltpu.sync_copy(acc_s, out.at[0, w])
    pltpu.sync_copy(acc_d, out.at[1, w])


def _make_deg_kernel():
    return pl.kernel(
        _deg_body,
        out_type=jax.ShapeDtypeStruct((2, NW, NP), jnp.float32),
        mesh=_mesh(),
        compiler_params=_sc_compiler_params(),
        scratch_types=[
            pltpu.VMEM((NP,), jnp.float32),
            pltpu.VMEM((NP,), jnp.float32),
            pltpu.VMEM((2, DEG_CHUNK), jnp.int32),
            pltpu.SemaphoreType.DMA,
            pltpu.SemaphoreType.DMA,
        ],
    )


UNIT = 64
UNITS_PER_TILE = ROWS_PER_TILE * 128 // UNIT
GUNITS = 16
SEG_GROUPS = UNITS_PER_TILE // GUNITS
ERU = ER * 128 // UNIT
NSLOT = 4


def _seg_body(h, srcr, dstr, out, acc, rows, idx, zb,
              g0, g1, g2, g3, s0, s1, s2, s3, i0, i1):
    c = lax.axis_index("c")
    s = lax.axis_index("s")
    w = c * NS + s
    z16 = jnp.zeros((16,), jnp.float32)
    gsems = (g0, g1, g2, g3)
    ssems = (s0, s1, s2, s3)
    isems = (i0, i1)

    @pl.loop(0, 40)
    def _(i):
        for k in range(8):
            zb[i, pl.ds(k * 16, 16)] = z16

    node0 = s * NODE_ROWS_PER_TILE
    zds = []
    for t in range(NODE_ROWS_PER_TILE // 40):
        zds.append(pltpu.async_copy(zb, acc.at[pl.ds(node0 + t * 40, 40)],
                                    gsems[t % 4]))
    for dz in zds:
        dz.wait()
    plsc.subcore_barrier()

    base = w * UNITS_PER_TILE

    def fetch_idx(g, b):
        r0 = jnp.minimum(base + g * GUNITS, ERU - GUNITS)
        return (pltpu.async_copy(srcr.at[pl.ds(r0, GUNITS)], idx.at[b, 0],
                                 isems[b]),
                pltpu.async_copy(dstr.at[pl.ds(r0, GUNITS)], idx.at[b, 1],
                                 isems[b]))

    for d0 in fetch_idx(0, 0):
        d0.wait()

    def process_group(g, b, prev_tail):
        nxt = fetch_idx(g + 1, 1 - b)
        gd = [None] * GUNITS
        sd = [None] * GUNITS
        for u in range(GUNITS):
            sl = u % NSLOT
            if u >= NSLOT:
                sd[u - NSLOT].wait()
            elif prev_tail is not None:
                prev_tail[u].wait()
            gd[u] = pltpu.async_copy(h.at[idx.at[b, 0, u]], rows.at[sl],
                                     gsems[sl])
            if u >= 2:
                gd[u - 2].wait()
                sd[u - 2] = pltpu.async_copy(rows.at[(u - 2) % NSLOT],
                                             acc.at[idx.at[b, 1, u - 2]],
                                             ssems[(u - 2) % NSLOT], add=True)
        for u in (GUNITS - 2, GUNITS - 1):
            gd[u].wait()
            sd[u] = pltpu.async_copy(rows.at[u % NSLOT],
                                     acc.at[idx.at[b, 1, u]],
                                     ssems[u % NSLOT], add=True)
        for d1 in nxt:
            d1.wait()
        return sd[GUNITS - NSLOT:]

    @pl.loop(0, SEG_GROUPS // 2)
    def _(t):
        tail0 = process_group(2 * t, 0, None)
        tail1 = process_group(2 * t + 1, 1, tail0)
        for d in tail1:
            d.wait()

    plsc.subcore_barrier()
    rds = []
    for t in range(NODE_ROWS_PER_TILE // 128):
        sl = pl.ds(node0 + t * 128, 128)
        rds.append(pltpu.async_copy(acc.at[sl], out.at[c, sl], gsems[t % 4]))
    for r in rds:
        r.wait()


def _make_seg_kernel():
    return pl.kernel(
        _seg_body,
        out_type=jax.ShapeDtypeStruct((NC, NP, D), jnp.float32),
        mesh=_mesh(),
        compiler_params=_sc_compiler_params(),
        scratch_types=[
            pltpu.VMEM_SHARED((NP, D), jnp.float32),
            pltpu.VMEM((NSLOT, UNIT, D), jnp.float32),
            pltpu.VMEM((2, 2, GUNITS, UNIT), jnp.int32),
            pltpu.VMEM((40, D), jnp.float32),
        ] + [pltpu.SemaphoreType.DMA] * 10,
    )


def _rt_scales(cnt):
    deg = jnp.maximum(jnp.sum(cnt, axis=1), 1.0)
    return jnp.transpose(lax.rsqrt(deg))


def _scales_body(cnt_ref, x_ref, si_ref, h1_ref):
    rt = _rt_scales(cnt_ref[...])
    ones_row = jnp.ones((1, D), jnp.float32)
    si_ref[...] = rt[:, 1:2] * ones_row
    h1_ref[...] = x_ref[...] * rt[:, 0:1]


_scales_call = pl.pallas_call(
    _scales_body,
    grid=(GRID,),
    in_specs=[pl.BlockSpec((2, NW, NBLK), lambda i: (0, 0, i)),
              pl.BlockSpec((NBLK, D), lambda i: (i, 0))],
    out_specs=[pl.BlockSpec((NBLK, D), lambda i: (i, 0)),
               pl.BlockSpec((NBLK, D), lambda i: (i, 0))],
    out_shape=[jax.ShapeDtypeStruct((NP, D), jnp.float32),
               jax.ShapeDtypeStruct((NP, D), jnp.float32)],
)


def _combine1_body(parts_ref, cnt_ref, w_ref, b_ref, o_ref):
    rt = _rt_scales(cnt_ref[...])
    pp = parts_ref[...]
    p = (pp[0] + pp[1]) * rt[:, 1:2]
    z = lax.dot_general(p, w_ref[...], (((1,), (0,)), ((), ())),
                        preferred_element_type=jnp.float32,
                        precision=lax.Precision.DEFAULT)
    z = z + b_ref[...]
    o_ref[...] = jnp.maximum(z, 0.0) * rt[:, 0:1]


def _combine2_body(parts_ref, si_ref, w_ref, b_ref, o_ref):
    pp = parts_ref[...]
    p = (pp[0] + pp[1]) * si_ref[...]
    z = lax.dot_general(p, w_ref[...], (((1,), (0,)), ((), ())),
                        preferred_element_type=jnp.float32,
                        precision=lax.Precision.DEFAULT)
    o_ref[...] = z + b_ref[...]


_nd_spec = pl.BlockSpec((NBLK, D), lambda i: (i, 0))

_combine1_call = pl.pallas_call(
    _combine1_body,
    grid=(GRID,),
    in_specs=[
        pl.BlockSpec((NC, NBLK, D), lambda i: (0, i, 0)),
        pl.BlockSpec((2, NW, NBLK), lambda i: (0, 0, i)),
        pl.BlockSpec((D, D), lambda i: (0, 0)),
        pl.BlockSpec((1, D), lambda i: (0, 0)),
    ],
    out_specs=_nd_spec,
    out_shape=jax.ShapeDtypeStruct((NP, D), jnp.float32),
)

NBLK2 = 1000
_combine2_call = pl.pallas_call(
    _combine2_body,
    grid=(N // NBLK2,),
    in_specs=[
        pl.BlockSpec((NC, NBLK2, D), lambda i: (0, i, 0)),
        pl.BlockSpec((NBLK2, D), lambda i: (i, 0)),
        pl.BlockSpec((D, D), lambda i: (0, 0)),
        pl.BlockSpec((1, D), lambda i: (0, 0)),
    ],
    out_specs=pl.BlockSpec((NBLK2, D), lambda i: (i, 0)),
    out_shape=jax.ShapeDtypeStruct((N, D), jnp.float32),
)


def kernel(x, edge_index, W1, b1, W2, b2):
    src = edge_index[0]
    dst = edge_index[1]
    padlen = ER * 128 - E
    padidx = (N + (jnp.arange(padlen, dtype=jnp.int32) % (NP - N))
              ).astype(jnp.int32)
    srcr = jnp.concatenate([src, padidx]).reshape(ERU, UNIT)
    dstr = jnp.concatenate([dst, padidx]).reshape(ERU, UNIT)
    x_pad = jnp.pad(x, ((0, NP - N), (0, 0)))
    b1r = b1.reshape(1, D)
    b2r = b2.reshape(1, D)

    cnt = _make_deg_kernel()(edge_index)
    si, h1 = _scales_call(cnt, x_pad)
    seg = _make_seg_kernel()
    parts1 = seg(h1, srcr, dstr)
    h2 = _combine1_call(parts1, cnt, W1, b1r)
    parts2 = seg(h2, srcr, dstr)
    return _combine2_call(parts2, si, W2, b2r)


# speedup vs baseline: 13.5800x; 1.0243x over previous
import dataclasses
import functools

import jax
import jax.numpy as jnp
from jax import lax
from jax.experimental import pallas as pl
from jax.experimental.pallas import tpu as pltpu
from jax.experimental.pallas import tpu_sc as plsc

N = 10000
NP = 10240
D = 128
E = 320000
ER = 2560
NC, NS = 2, 16
NW = NC * NS
ROWS_PER_TILE = ER // NW
GROUPS = ROWS_PER_TILE // 8
NODE_ROWS_PER_TILE = NP // NS
NBLK = 2048
GRID = NP // NBLK


def _mesh():
    return plsc.VectorSubcoreMesh(core_axis_name="c", subcore_axis_name="s")


def _sc_compiler_params():
    cp = pltpu.CompilerParams()
    if "needs_layout_passes" in pltpu.CompilerParams.__dataclass_fields__:
        cp = dataclasses.replace(cp, needs_layout_passes=False)
    return cp


DEG_CHUNK = 2560
DEG_NGRP = E // DEG_CHUNK
DEG_ITERS = -(-DEG_NGRP // NW)


def _deg_body(ei, out, acc_s, acc_d, idx, i0, i1):
    c = lax.axis_index("c")
    s = lax.axis_index("s")
    w = c * NS + s
    z16 = jnp.zeros((16,), jnp.float32)
    isems = (i0, i1)

    @pl.loop(0, NP // 16)
    def _(i):
        acc_s[pl.ds(i * 16, 16)] = z16
        acc_d[pl.ds(i * 16, 16)] = z16

    ones = jnp.ones((16,), jnp.float32)

    def fetch(h, i, b):
        g = jnp.minimum(w + NW * i, DEG_NGRP - 1)
        return pltpu.async_copy(ei.at[h, pl.ds(g * DEG_CHUNK, DEG_CHUNK)],
                                idx.at[b], isems[b])

    for h, acc in ((0, acc_s), (1, acc_d)):
        fetch(h, 0, 0).wait()

        def body(i, b, acc=acc, h=h):
            nxt = fetch(h, i + 1, 1 - b)

            @pl.when(w + NW * i < DEG_NGRP)
            def _():
                @pl.loop(0, DEG_CHUNK // 16)
                def _(n):
                    iv = idx[b, pl.ds(n * 16, 16)]
                    plsc.addupdate_scatter(acc, [iv], ones)

            nxt.wait()

        @pl.loop(0, DEG_ITERS // 2)
        def _(t):
            body(2 * t, 0)
            body(2 * t + 1, 1)

    pltpu.sync_copy(acc_s, out.at[0, w])
    pltpu.sync_copy(acc_d, out.at[1, w])


def _make_deg_kernel():
    return pl.kernel(
        _deg_body,
        out_type=jax.ShapeDtypeStruct((2, NW, NP), jnp.float32),
        mesh=_mesh(),
        compiler_params=_sc_compiler_params(),
        scratch_types=[
            pltpu.VMEM((NP,), jnp.float32),
            pltpu.VMEM((NP,), jnp.float32),
            pltpu.VMEM((2, DEG_CHUNK), jnp.int32),
            pltpu.SemaphoreType.DMA,
            pltpu.SemaphoreType.DMA,
        ],
    )


UNIT = 64
UNITS_PER_TILE = ROWS_PER_TILE * 128 // UNIT
GUNITS = 16
SEG_GROUPS = UNITS_PER_TILE // GUNITS
ERU = ER * 128 // UNIT
NSLOT = 4


def _seg_body(h, srcr, dstr, out, acc, rows, idx, zb,
              g0, g1, g2, g3, s0, s1, s2, s3, i0, i1):
    c = lax.axis_index("c")
    s = lax.axis_index("s")
    w = c * NS + s
    z16 = jnp.zeros((16,), jnp.float32)
    gsems = (g0, g1, g2, g3)
    ssems = (s0, s1, s2, s3)
    isems = (i0, i1)

    @pl.loop(0, 40)
    def _(i):
        for k in range(8):
            zb[i, pl.ds(k * 16, 16)] = z16

    node0 = s * NODE_ROWS_PER_TILE
    zds = []
    for t in range(NODE_ROWS_PER_TILE // 40):
        zds.append(pltpu.async_copy(zb, acc.at[pl.ds(node0 + t * 40, 40)],
                                    gsems[t % 4]))
    for dz in zds:
        dz.wait()
    plsc.subcore_barrier()

    base = w * UNITS_PER_TILE

    def fetch_idx(g, b):
        r0 = jnp.minimum(base + g * GUNITS, ERU - GUNITS)
        return (pltpu.async_copy(srcr.at[pl.ds(r0, GUNITS)], idx.at[b, 0],
                                 isems[b]),
                pltpu.async_copy(dstr.at[pl.ds(r0, GUNITS)], idx.at[b, 1],
                                 isems[b]))

    for d0 in fetch_idx(0, 0):
        d0.wait()

    def process_group(g, b, prev_tail):
        nxt = fetch_idx(g + 1, 1 - b)
        gd = [None] * GUNITS
        sd = [None] * GUNITS
        for u in range(GUNITS):
            sl = u % NSLOT
            if u >= NSLOT:
                sd[u - NSLOT].wait()
            elif prev_tail is not None:
                prev_tail[u].wait()
            gd[u] = pltpu.async_copy(h.at[idx.at[b, 0, u]], rows.at[sl],
                                     gsems[sl])
            if u >= 2:
                gd[u - 2].wait()
                sd[u - 2] = pltpu.async_copy(rows.at[(u - 2) % NSLOT],
                                             acc.at[idx.at[b, 1, u - 2]],
                                             ssems[(u - 2) % NSLOT], add=True)
        for u in (GUNITS - 2, GUNITS - 1):
            gd[u].wait()
            sd[u] = pltpu.async_copy(rows.at[u % NSLOT],
                                     acc.at[idx.at[b, 1, u]],
                                     ssems[u % NSLOT], add=True)
        for d1 in nxt:
            d1.wait()
        return sd[GUNITS - NSLOT:]

    @pl.loop(0, SEG_GROUPS // 2)
    def _(t):
        tail0 = process_group(2 * t, 0, None)
        tail1 = process_group(2 * t + 1, 1, tail0)
        for d in tail1:
            d.wait()

    plsc.subcore_barrier()
    rds = []
    for t in range(NODE_ROWS_PER_TILE // 128):
        sl = pl.ds(node0 + t * 128, 128)
        rds.append(pltpu.async_copy(acc.at[sl], out.at[c, sl], gsems[t % 4]))
    for r in rds:
        r.wait()


def _make_seg_kernel():
    return pl.kernel(
        _seg_body,
        out_type=jax.ShapeDtypeStruct((NC, NP, D), jnp.float32),
        mesh=_mesh(),
        compiler_params=_sc_compiler_params(),
        scratch_types=[
            pltpu.VMEM_SHARED((NP, D), jnp.float32),
            pltpu.VMEM((NSLOT, UNIT, D), jnp.float32),
            pltpu.VMEM((2, 2, GUNITS, UNIT), jnp.int32),
            pltpu.VMEM((40, D), jnp.float32),
        ] + [pltpu.SemaphoreType.DMA] * 10,
    )


def _rt_scales(cnt):
    deg = jnp.maximum(jnp.sum(cnt, axis=1), 1.0)
    return jnp.transpose(lax.rsqrt(deg))


def _scales_body(cnt_ref, x_ref, si_ref, h1_ref):
    rt = _rt_scales(cnt_ref[...])
    ones_row = jnp.ones((1, D), jnp.float32)
    si_ref[...] = rt[:, 1:2] * ones_row
    h1_ref[...] = x_ref[...] * rt[:, 0:1]


_scales_call = pl.pallas_call(
    _scales_body,
    grid=(GRID,),
    in_specs=[pl.BlockSpec((2, NW, NBLK), lambda i: (0, 0, i)),
              pl.BlockSpec((NBLK, D), lambda i: (i, 0))],
    out_specs=[pl.BlockSpec((NBLK, D), lambda i: (i, 0)),
               pl.BlockSpec((NBLK, D), lambda i: (i, 0))],
    out_shape=[jax.ShapeDtypeStruct((NP, D), jnp.float32),
               jax.ShapeDtypeStruct((NP, D), jnp.float32)],
)


def _combine1_body(parts_ref, cnt_ref, w_ref, b_ref, o_ref):
    rt = _rt_scales(cnt_ref[...])
    pp = parts_ref[...]
    p = (pp[0] + pp[1]) * rt[:, 1:2]
    z = lax.dot_general(p, w_ref[...], (((1,), (0,)), ((), ())),
                        preferred_element_type=jnp.float32,
                        precision=lax.Precision.DEFAULT)
    z = z + b_ref[...]
    o_ref[...] = jnp.maximum(z, 0.0) * rt[:, 0:1]


def _combine2_body(parts_ref, si_ref, w_ref, b_ref, o_ref):
    pp = parts_ref[...]
    p = (pp[0] + pp[1]) * si_ref[...]
    z = lax.dot_general(p, w_ref[...], (((1,), (0,)), ((), ())),
                        preferred_element_type=jnp.float32,
                        precision=lax.Precision.DEFAULT)
    o_ref[...] = z + b_ref[...]


_nd_spec = pl.BlockSpec((NBLK, D), lambda i: (i, 0))

_combine1_call = pl.pallas_call(
    _combine1_body,
    grid=(GRID,),
    in_specs=[
        pl.BlockSpec((NC, NBLK, D), lambda i: (0, i, 0)),
        pl.BlockSpec((2, NW, NBLK), lambda i: (0, 0, i)),
        pl.BlockSpec((D, D), lambda i: (0, 0)),
        pl.BlockSpec((1, D), lambda i: (0, 0)),
    ],
    out_specs=_nd_spec,
    out_shape=jax.ShapeDtypeStruct((NP, D), jnp.float32),
)

NBLK2 = 2000
_combine2_call = pl.pallas_call(
    _combine2_body,
    grid=(N // NBLK2,),
    in_specs=[
        pl.BlockSpec((NC, NBLK2, D), lambda i: (0, i, 0)),
        pl.BlockSpec((NBLK2, D), lambda i: (i, 0)),
        pl.BlockSpec((D, D), lambda i: (0, 0)),
        pl.BlockSpec((1, D), lambda i: (0, 0)),
    ],
    out_specs=pl.BlockSpec((NBLK2, D), lambda i: (i, 0)),
    out_shape=jax.ShapeDtypeStruct((N, D), jnp.float32),
)


def kernel(x, edge_index, W1, b1, W2, b2):
    src = edge_index[0]
    dst = edge_index[1]
    padlen = ER * 128 - E
    padidx = (N + (jnp.arange(padlen, dtype=jnp.int32) % (NP - N))
              ).astype(jnp.int32)
    srcr = jnp.concatenate([src, padidx]).reshape(ERU, UNIT)
    dstr = jnp.concatenate([dst, padidx]).reshape(ERU, UNIT)
    x_pad = jnp.pad(x, ((0, NP - N), (0, 0)))
    b1r = b1.reshape(1, D)
    b2r = b2.reshape(1, D)

    cnt = _make_deg_kernel()(edge_index)
    si, h1 = _scales_call(cnt, x_pad)
    seg = _make_seg_kernel()
    parts1 = seg(h1, srcr, dstr)
    h2 = _combine1_call(parts1, cnt, W1, b1r)
    parts2 = seg(h2, srcr, dstr)
    return _combine2_call(parts2, si, W2, b2r)


# speedup vs baseline: 13.7963x; 1.0159x over previous
import dataclasses
import functools

import jax
import jax.numpy as jnp
from jax import lax
from jax.experimental import pallas as pl
from jax.experimental.pallas import tpu as pltpu
from jax.experimental.pallas import tpu_sc as plsc

N = 10000
NP = 10240
D = 128
E = 320000
ER = 2560
NC, NS = 2, 16
NW = NC * NS
ROWS_PER_TILE = ER // NW
GROUPS = ROWS_PER_TILE // 8
NODE_ROWS_PER_TILE = NP // NS
NBLK = 5120
GRID = NP // NBLK


def _mesh():
    return plsc.VectorSubcoreMesh(core_axis_name="c", subcore_axis_name="s")


def _sc_compiler_params():
    cp = pltpu.CompilerParams()
    if "needs_layout_passes" in pltpu.CompilerParams.__dataclass_fields__:
        cp = dataclasses.replace(cp, needs_layout_passes=False)
    return cp


DEG_CHUNK = 2560
DEG_NGRP = E // DEG_CHUNK
DEG_ITERS = -(-DEG_NGRP // NW)


def _deg_body(ei, out, acc_s, acc_d, idx, i0, i1):
    c = lax.axis_index("c")
    s = lax.axis_index("s")
    w = c * NS + s
    z16 = jnp.zeros((16,), jnp.float32)
    isems = (i0, i1)

    @pl.loop(0, NP // 16)
    def _(i):
        acc_s[pl.ds(i * 16, 16)] = z16
        acc_d[pl.ds(i * 16, 16)] = z16

    ones = jnp.ones((16,), jnp.float32)

    def fetch(h, i, b):
        g = jnp.minimum(w + NW * i, DEG_NGRP - 1)
        return pltpu.async_copy(ei.at[h, pl.ds(g * DEG_CHUNK, DEG_CHUNK)],
                                idx.at[b], isems[b])

    for h, acc in ((0, acc_s), (1, acc_d)):
        fetch(h, 0, 0).wait()

        def body(i, b, acc=acc, h=h):
            nxt = fetch(h, i + 1, 1 - b)

            @pl.when(w + NW * i < DEG_NGRP)
            def _():
                @pl.loop(0, DEG_CHUNK // 16)
                def _(n):
                    iv = idx[b, pl.ds(n * 16, 16)]
                    plsc.addupdate_scatter(acc, [iv], ones)

            nxt.wait()

        @pl.loop(0, DEG_ITERS // 2)
        def _(t):
            body(2 * t, 0)
            body(2 * t + 1, 1)

    pltpu.sync_copy(acc_s, out.at[0, w])
    pltpu.sync_copy(acc_d, out.at[1, w])


def _make_deg_kernel():
    return pl.kernel(
        _deg_body,
        out_type=jax.ShapeDtypeStruct((2, NW, NP), jnp.float32),
        mesh=_mesh(),
        compiler_params=_sc_compiler_params(),
        scratch_types=[
            pltpu.VMEM((NP,), jnp.float32),
            pltpu.VMEM((NP,), jnp.float32),
            pltpu.VMEM((2, DEG_CHUNK), jnp.int32),
            pltpu.SemaphoreType.DMA,
            pltpu.SemaphoreType.DMA,
        ],
    )


UNIT = 64
UNITS_PER_TILE = ROWS_PER_TILE * 128 // UNIT
GUNITS = 16
SEG_GROUPS = UNITS_PER_TILE // GUNITS
ERU = ER * 128 // UNIT
NSLOT = 4


def _seg_body(h, srcr, dstr, out, acc, rows, idx, zb,
              g0, g1, g2, g3, s0, s1, s2, s3, i0, i1):
    c = lax.axis_index("c")
    s = lax.axis_index("s")
    w = c * NS + s
    z16 = jnp.zeros((16,), jnp.float32)
    gsems = (g0, g1, g2, g3)
    ssems = (s0, s1, s2, s3)
    isems = (i0, i1)

    @pl.loop(0, 40)
    def _(i):
        for k in range(8):
            zb[i, pl.ds(k * 16, 16)] = z16

    node0 = s * NODE_ROWS_PER_TILE
    zds = []
    for t in range(NODE_ROWS_PER_TILE // 40):
        zds.append(pltpu.async_copy(zb, acc.at[pl.ds(node0 + t * 40, 40)],
                                    gsems[t % 4]))
    for dz in zds:
        dz.wait()
    plsc.subcore_barrier()

    base = w * UNITS_PER_TILE

    def fetch_idx(g, b):
        r0 = jnp.minimum(base + g * GUNITS, ERU - GUNITS)
        return (pltpu.async_copy(srcr.at[pl.ds(r0, GUNITS)], idx.at[b, 0],
                                 isems[b]),
                pltpu.async_copy(dstr.at[pl.ds(r0, GUNITS)], idx.at[b, 1],
                                 isems[b]))

    for d0 in fetch_idx(0, 0):
        d0.wait()

    def process_group(g, b, prev_tail):
        nxt = fetch_idx(g + 1, 1 - b)
        gd = [None] * GUNITS
        sd = [None] * GUNITS
        for u in range(GUNITS):
            sl = u % NSLOT
            if u >= NSLOT:
                sd[u - NSLOT].wait()
            elif prev_tail is not None:
                prev_tail[u].wait()
            gd[u] = pltpu.async_copy(h.at[idx.at[b, 0, u]], rows.at[sl],
                                     gsems[sl])
            if u >= 2:
                gd[u - 2].wait()
                sd[u - 2] = pltpu.async_copy(rows.at[(u - 2) % NSLOT],
                                             acc.at[idx.at[b, 1, u - 2]],
                                             ssems[(u - 2) % NSLOT], add=True)
        for u in (GUNITS - 2, GUNITS - 1):
            gd[u].wait()
            sd[u] = pltpu.async_copy(rows.at[u % NSLOT],
                                     acc.at[idx.at[b, 1, u]],
                                     ssems[u % NSLOT], add=True)
        for d1 in nxt:
            d1.wait()
        return sd[GUNITS - NSLOT:]

    @pl.loop(0, SEG_GROUPS // 2)
    def _(t):
        tail0 = process_group(2 * t, 0, None)
        tail1 = process_group(2 * t + 1, 1, tail0)
        for d in tail1:
            d.wait()

    plsc.subcore_barrier()
    rds = []
    for t in range(NODE_ROWS_PER_TILE // 128):
        sl = pl.ds(node0 + t * 128, 128)
        rds.append(pltpu.async_copy(acc.at[sl], out.at[c, sl], gsems[t % 4]))
    for r in rds:
        r.wait()


def _make_seg_kernel():
    return pl.kernel(
        _seg_body,
        out_type=jax.ShapeDtypeStruct((NC, NP, D), jnp.float32),
        mesh=_mesh(),
        compiler_params=_sc_compiler_params(),
        scratch_types=[
            pltpu.VMEM_SHARED((NP, D), jnp.float32),
            pltpu.VMEM((NSLOT, UNIT, D), jnp.float32),
            pltpu.VMEM((2, 2, GUNITS, UNIT), jnp.int32),
            pltpu.VMEM((40, D), jnp.float32),
        ] + [pltpu.SemaphoreType.DMA] * 10,
    )


def _rt_scales(cnt):
    deg = jnp.maximum(jnp.sum(cnt, axis=1), 1.0)
    return jnp.transpose(lax.rsqrt(deg))


def _scales_body(cnt_ref, x_ref, si_ref, h1_ref):
    rt = _rt_scales(cnt_ref[...])
    ones_row = jnp.ones((1, D), jnp.float32)
    si_ref[...] = rt[:, 1:2] * ones_row
    h1_ref[...] = x_ref[...] * rt[:, 0:1]


_scales_call = pl.pallas_call(
    _scales_body,
    grid=(GRID,),
    in_specs=[pl.BlockSpec((2, NW, NBLK), lambda i: (0, 0, i)),
              pl.BlockSpec((NBLK, D), lambda i: (i, 0))],
    out_specs=[pl.BlockSpec((NBLK, D), lambda i: (i, 0)),
               pl.BlockSpec((NBLK, D), lambda i: (i, 0))],
    out_shape=[jax.ShapeDtypeStruct((NP, D), jnp.float32),
               jax.ShapeDtypeStruct((NP, D), jnp.float32)],
)


def _combine1_body(parts_ref, cnt_ref, w_ref, b_ref, o_ref):
    rt = _rt_scales(cnt_ref[...])
    pp = parts_ref[...]
    p = (pp[0] + pp[1]) * rt[:, 1:2]
    z = lax.dot_general(p, w_ref[...], (((1,), (0,)), ((), ())),
                        preferred_element_type=jnp.float32,
                        precision=lax.Precision.DEFAULT)
    z = z + b_ref[...]
    o_ref[...] = jnp.maximum(z, 0.0) * rt[:, 0:1]


def _combine2_body(parts_ref, si_ref, w_ref, b_ref, o_ref):
    pp = parts_ref[...]
    p = (pp[0] + pp[1]) * si_ref[...]
    z = lax.dot_general(p, w_ref[...], (((1,), (0,)), ((), ())),
                        preferred_element_type=jnp.float32,
                        precision=lax.Precision.DEFAULT)
    o_ref[...] = z + b_ref[...]


_nd_spec = pl.BlockSpec((NBLK, D), lambda i: (i, 0))

_combine1_call = pl.pallas_call(
    _combine1_body,
    grid=(GRID,),
    in_specs=[
        pl.BlockSpec((NC, NBLK, D), lambda i: (0, i, 0)),
        pl.BlockSpec((2, NW, NBLK), lambda i: (0, 0, i)),
        pl.BlockSpec((D, D), lambda i: (0, 0)),
        pl.BlockSpec((1, D), lambda i: (0, 0)),
    ],
    out_specs=_nd_spec,
    out_shape=jax.ShapeDtypeStruct((NP, D), jnp.float32),
)

NBLK2 = 5000
_combine2_call = pl.pallas_call(
    _combine2_body,
    grid=(N // NBLK2,),
    in_specs=[
        pl.BlockSpec((NC, NBLK2, D), lambda i: (0, i, 0)),
        pl.BlockSpec((NBLK2, D), lambda i: (i, 0)),
        pl.BlockSpec((D, D), lambda i: (0, 0)),
        pl.BlockSpec((1, D), lambda i: (0, 0)),
    ],
    out_specs=pl.BlockSpec((NBLK2, D), lambda i: (i, 0)),
    out_shape=jax.ShapeDtypeStruct((N, D), jnp.float32),
)


def kernel(x, edge_index, W1, b1, W2, b2):
    src = edge_index[0]
    dst = edge_index[1]
    padlen = ER * 128 - E
    padidx = (N + (jnp.arange(padlen, dtype=jnp.int32) % (NP - N))
              ).astype(jnp.int32)
    srcr = jnp.concatenate([src, padidx]).reshape(ERU, UNIT)
    dstr = jnp.concatenate([dst, padidx]).reshape(ERU, UNIT)
    x_pad = jnp.pad(x, ((0, NP - N), (0, 0)))
    b1r = b1.reshape(1, D)
    b2r = b2.reshape(1, D)

    cnt = _make_deg_kernel()(edge_index)
    si, h1 = _scales_call(cnt, x_pad)
    seg = _make_seg_kernel()
    parts1 = seg(h1, srcr, dstr)
    h2 = _combine1_call(parts1, cnt, W1, b1r)
    parts2 = seg(h2, srcr, dstr)
    return _combine2_call(parts2, si, W2, b2r)
